# Initial kernel scaffold; baseline (speedup 1.0000x reference)
import jax
import jax.numpy as jnp
from jax.experimental import pallas as pl


def kernel(atom_repr, bond_repr, atom_nbr_d1, atom_nbr_d2, atom_nbr_d3, atom_nbr_d4, bond_nbr_d1, bond_nbr_d2, bond_nbr_d3, bond_nbr_d4, mol_ids, W_deg, W_self, conv_bias, W_out, b_out):
    raise NotImplementedError("write your pallas kernel here")


# speedup vs baseline: 3.0242x; 3.0242x over previous
import jax
import jax.numpy as jnp
from jax import lax
from jax.experimental import pallas as pl
from jax.experimental.pallas import tpu as pltpu
from jax.experimental.pallas import tpu_sc as plsc

N_ATOM = 100000
ND = 25000
NB = 250000
M_MOL = 4096
NODE = 128
EDGE = 16
OUT = 128
NL = 2
DEGS = (1, 2, 3, 4)

NTILE = 32
T_MAIN = 776
C_MAIN = 96
N_CHUNK = 8
T_TAIL = 8
REM_TILES = 21
GOFF = (0, ND, 3 * ND, 6 * ND)


def _conv_body(x_hbm, aidx_hbm, a_out,
               idxa0, idxa1, idxa2, idxa3, stga, gsem, osem):
    idxa = (idxa0, idxa1, idxa2, idxa3)
    cid = lax.axis_index("c")
    sid = lax.axis_index("s")
    w = sid * 2 + cid

    for g, d in enumerate(DEGS):
        goff = GOFF[g]
        orow0 = g * ND

        for j in range(d):
            pltpu.sync_copy(aidx_hbm.at[pl.ds(goff + j * ND + w * T_MAIN, T_MAIN)],
                            idxa[j].at[pl.ds(0, T_MAIN)])

        def chunk(col0, nrow, out_r, d=d):
            cps = []
            for j in range(d):
                cps.append(pltpu.async_copy(
                    x_hbm.at[idxa[j].at[pl.ds(col0, nrow)]],
                    stga.at[j, pl.ds(0, nrow), :], gsem))
            for cp in cps:
                cp.wait()
            if d > 1:
                def srow(r, _):
                    for col in range(NODE // 16):
                        sl = pl.ds(col * 16, 16)
                        v = stga[0, r, sl]
                        for j in range(1, d):
                            v = v + stga[j, r, sl]
                        stga[0, r, sl] = v
                    return 0
                lax.fori_loop(0, nrow, srow, 0, unroll=False)
            pltpu.async_copy(stga.at[0, pl.ds(0, nrow), :],
                             a_out.at[pl.ds(out_r, nrow), :], osem).wait()

        row_base = orow0 + w * T_MAIN

        def main(k, _):
            chunk(k * C_MAIN, C_MAIN, row_base + k * C_MAIN)
            return 0
        lax.fori_loop(0, N_CHUNK, main, 0, unroll=False)
        chunk(N_CHUNK * C_MAIN, T_TAIL, row_base + N_CHUNK * C_MAIN)

        @pl.when(w < REM_TILES)
        def _():
            rbase = NTILE * T_MAIN
            for j in range(d):
                pltpu.sync_copy(
                    aidx_hbm.at[pl.ds(goff + j * ND + rbase + w * 8, 8)],
                    idxa[j].at[pl.ds(T_MAIN, 8)])
            chunk(T_MAIN, 8, orow0 + rbase + w * 8)


def _conv_gather(x, aidx):
    mesh = plsc.VectorSubcoreMesh(core_axis_name="c", subcore_axis_name="s",
                                  num_cores=2, num_subcores=16)
    f = pl.kernel(
        _conv_body,
        out_type=jax.ShapeDtypeStruct((N_ATOM, NODE), jnp.float32),
        mesh=mesh,
        scratch_types=(
            [pltpu.VMEM((T_MAIN + 8,), jnp.int32) for _ in range(4)]
            + [
                pltpu.VMEM((4, C_MAIN, NODE), jnp.float32),
                pltpu.SemaphoreType.DMA,
                pltpu.SemaphoreType.DMA,
            ]),
    )
    return f(x, aidx)


def _bond_body(bond_hbm, bidx_hbm, bf_out,
               idxb0, idxb1, idxb2, idxb3, stgb, bacc, gsem, osem):
    idxb = (idxb0, idxb1, idxb2, idxb3)
    cid = lax.axis_index("c")
    sid = lax.axis_index("s")
    w = sid * 2 + cid

    for g, d in enumerate(DEGS):
        goff = GOFF[g]
        orow0 = g * ND

        for j in range(d):
            pltpu.sync_copy(bidx_hbm.at[pl.ds(goff + j * ND + w * T_MAIN, T_MAIN)],
                            idxb[j].at[pl.ds(0, T_MAIN)])

        def chunk(col0, nrow, out_r, d=d):
            cps = []
            for j in range(d):
                cps.append(pltpu.async_copy(
                    bond_hbm.at[idxb[j].at[pl.ds(col0, nrow)]],
                    stgb.at[j, pl.ds(0, nrow), :], gsem))
            for cp in cps:
                cp.wait()

            def srow(r, _):
                v = stgb[0, r, :]
                for j in range(1, d):
                    v = v + stgb[j, r, :]
                bacc[pl.ds(r * EDGE, EDGE)] = v
                return 0
            lax.fori_loop(0, nrow, srow, 0, unroll=False)

            pltpu.async_copy(bacc.at[pl.ds(0, nrow * EDGE)],
                             bf_out.at[pl.ds(out_r * EDGE, nrow * EDGE)],
                             osem).wait()

        row_base = orow0 + w * T_MAIN

        def main(k, _):
            chunk(k * C_MAIN, C_MAIN, row_base + k * C_MAIN)
            return 0
        lax.fori_loop(0, N_CHUNK, main, 0, unroll=False)
        chunk(N_CHUNK * C_MAIN, T_TAIL, row_base + N_CHUNK * C_MAIN)

        @pl.when(w < REM_TILES)
        def _():
            rbase = NTILE * T_MAIN
            for j in range(d):
                pltpu.sync_copy(
                    bidx_hbm.at[pl.ds(goff + j * ND + rbase + w * 8, 8)],
                    idxb[j].at[pl.ds(T_MAIN, 8)])
            chunk(T_MAIN, 8, orow0 + rbase + w * 8)


def _bond_gather(bond, bidx):
    mesh = plsc.VectorSubcoreMesh(core_axis_name="c", subcore_axis_name="s",
                                  num_cores=2, num_subcores=16)
    f = pl.kernel(
        _bond_body,
        out_type=jax.ShapeDtypeStruct((N_ATOM * EDGE,), jnp.float32),
        mesh=mesh,
        scratch_types=(
            [pltpu.VMEM((T_MAIN + 8,), jnp.int32) for _ in range(4)]
            + [
                pltpu.VMEM((4, C_MAIN, EDGE), jnp.float32),
                pltpu.VMEM((C_MAIN * EDGE,), jnp.float32),
                pltpu.SemaphoreType.DMA,
                pltpu.SemaphoreType.DMA,
            ]),
        compiler_params=pltpu.CompilerParams(use_tc_tiling_on_sc=False),
    )
    return f(bond, bidx)


SEG_T = 3128
SEG_C = 128


def _seg_body(s_hbm, ids_hbm, out_hbm, stg, idxm, idx56, idx8, zbuf, acc, sem):
    cid = lax.axis_index("c")
    sid = lax.axis_index("s")

    def zrow(r, _):
        for col in range(NODE // 16):
            zbuf[r, pl.ds(col * 16, 16)] = jnp.zeros((16,), jnp.float32)
        return 0
    lax.fori_loop(0, M_MOL // 16, zrow, 0, unroll=False)
    pltpu.sync_copy(zbuf, acc.at[pl.ds(sid * (M_MOL // 16), M_MOL // 16), :])
    plsc.subcore_barrier()

    base = cid * (N_ATOM // 2) + sid * SEG_T

    def chunk(k, _):
        r0 = base + k * SEG_C
        pltpu.sync_copy(ids_hbm.at[pl.ds(r0, SEG_C)], idxm)
        pltpu.sync_copy(s_hbm.at[pl.ds(r0, SEG_C), :], stg)
        pltpu.sync_copy(stg, acc.at[idxm], add=True)
        return 0
    lax.fori_loop(0, 24, chunk, 0, unroll=False)

    r0 = base + 24 * SEG_C

    @pl.when(sid < 15)
    def _():
        pltpu.sync_copy(ids_hbm.at[pl.ds(r0, 56)], idx56)
        pltpu.sync_copy(s_hbm.at[pl.ds(r0, 56), :], stg.at[pl.ds(0, 56), :])
        pltpu.sync_copy(stg.at[pl.ds(0, 56), :], acc.at[idx56], add=True)

    @pl.when(sid == 15)
    def _():
        pltpu.sync_copy(ids_hbm.at[pl.ds(r0, 8)], idx8)
        pltpu.sync_copy(s_hbm.at[pl.ds(r0, 8), :], stg.at[pl.ds(0, 8), :])
        pltpu.sync_copy(stg.at[pl.ds(0, 8), :], acc.at[idx8], add=True)

    plsc.subcore_barrier()
    pltpu.sync_copy(acc.at[pl.ds(sid * 256, 256), :],
                    out_hbm.at[cid, pl.ds(sid * 256, 256), :])


def _segment_scatter(s, mol_ids):
    mesh = plsc.VectorSubcoreMesh(core_axis_name="c", subcore_axis_name="s",
                                  num_cores=2, num_subcores=16)
    f = pl.kernel(
        _seg_body,
        out_type=jax.ShapeDtypeStruct((2, M_MOL, OUT), jnp.float32),
        mesh=mesh,
        scratch_types=[
            pltpu.VMEM((SEG_C, OUT), jnp.float32),
            pltpu.VMEM((SEG_C,), jnp.int32),
            pltpu.VMEM((56,), jnp.int32),
            pltpu.VMEM((8,), jnp.int32),
            pltpu.VMEM((M_MOL // 16, OUT), jnp.float32),
            pltpu.VMEM_SHARED((M_MOL, OUT), jnp.float32),
            pltpu.SemaphoreType.DMA,
        ],
    )
    return f(s, mol_ids)


BLK = 1000
NBLK = N_ATOM // BLK
BPG = ND // BLK


def _layer_body(x_ref, a_ref, bf_ref, wda_ref, kb_ref, ws_ref, wo_ref,
                bo_ref, cb_ref, s_ref, act_ref, st_ref):
    i = pl.program_id(0)
    x = x_ref[...]
    y = jnp.dot(x, wo_ref[...], preferred_element_type=jnp.float32) + bo_ref[0][None, :]
    y = y - jnp.max(y, axis=1, keepdims=True)
    e = jnp.exp(y)
    s_ref[...] = e / jnp.sum(e, axis=1, keepdims=True)

    bf = bf_ref[...].reshape(BLK // 8, NODE)
    bp = jnp.dot(bf, kb_ref[0], preferred_element_type=jnp.float32)
    act = (jnp.dot(x, ws_ref[...], preferred_element_type=jnp.float32)
           + jnp.dot(a_ref[...], wda_ref[0], preferred_element_type=jnp.float32)
           + bp.reshape(BLK, OUT)
           + cb_ref[0][None, :])
    act_ref[...] = act

    @pl.when(i == 0)
    def _():
        st_ref[...] = jnp.zeros_like(st_ref)
    st_ref[0, :] = st_ref[0, :] + jnp.sum(act, axis=0)
    st_ref[1, :] = st_ref[1, :] + jnp.sum(act * act, axis=0)


def _layer_tc(x, a_sum, bfold, wda, kb, wself, wout, bout, cbias):
    return pl.pallas_call(
        _layer_body,
        grid=(NBLK,),
        in_specs=[
            pl.BlockSpec((BLK, NODE), lambda i: (i, 0)),
            pl.BlockSpec((BLK, NODE), lambda i: (i, 0)),
            pl.BlockSpec((BLK // 8, 1, NODE), lambda i: (i, 0, 0)),
            pl.BlockSpec((1, NODE, OUT), lambda i: (i // BPG, 0, 0)),
            pl.BlockSpec((1, NODE, 8 * OUT), lambda i: (i // BPG, 0, 0)),
            pl.BlockSpec((NODE, OUT), lambda i: (0, 0)),
            pl.BlockSpec((NODE, OUT), lambda i: (0, 0)),
            pl.BlockSpec((1, OUT), lambda i: (0, 0)),
            pl.BlockSpec((1, OUT), lambda i: (0, 0)),
        ],
        out_specs=[
            pl.BlockSpec((BLK, OUT), lambda i: (i, 0)),
            pl.BlockSpec((BLK, OUT), lambda i: (i, 0)),
            pl.BlockSpec((8, OUT), lambda i: (0, 0)),
        ],
        out_shape=[
            jax.ShapeDtypeStruct((N_ATOM, OUT), jnp.float32),
            jax.ShapeDtypeStruct((N_ATOM, OUT), jnp.float32),
            jax.ShapeDtypeStruct((8, OUT), jnp.float32),
        ],
        compiler_params=pltpu.CompilerParams(dimension_semantics=("arbitrary",)),
    )(x, a_sum, bfold, wda, kb, wself, wout, bout, cbias)


def _fp_body(x_ref, wo_ref, bo_ref, s_ref):
    y = (jnp.dot(x_ref[...], wo_ref[...], preferred_element_type=jnp.float32)
         + bo_ref[0][None, :])
    y = y - jnp.max(y, axis=1, keepdims=True)
    e = jnp.exp(y)
    s_ref[...] = e / jnp.sum(e, axis=1, keepdims=True)


def _fp_tc(x, wout, bout):
    return pl.pallas_call(
        _fp_body,
        grid=(NBLK,),
        in_specs=[
            pl.BlockSpec((BLK, NODE), lambda i: (i, 0)),
            pl.BlockSpec((NODE, OUT), lambda i: (0, 0)),
            pl.BlockSpec((1, OUT), lambda i: (0, 0)),
        ],
        out_specs=pl.BlockSpec((BLK, OUT), lambda i: (i, 0)),
        out_shape=jax.ShapeDtypeStruct((N_ATOM, OUT), jnp.float32),
    )(x, wout, bout)


def _norm_body(act_ref, st_ref, x_ref):
    mean = st_ref[0, :] * (1.0 / N_ATOM)
    var = st_ref[1, :] * (1.0 / N_ATOM) - mean * mean
    inv = lax.rsqrt(var + 1e-5)
    x_ref[...] = jnp.maximum((act_ref[...] - mean[None, :]) * inv[None, :], 0.0)


def _norm_tc(act, st):
    return pl.pallas_call(
        _norm_body,
        grid=(NBLK,),
        in_specs=[
            pl.BlockSpec((BLK, OUT), lambda i: (i, 0)),
            pl.BlockSpec((8, OUT), lambda i: (0, 0)),
        ],
        out_specs=pl.BlockSpec((BLK, OUT), lambda i: (i, 0)),
        out_shape=jax.ShapeDtypeStruct((N_ATOM, OUT), jnp.float32),
    )(act, st)


def _fpsum_body(p0_ref, p1_ref, p2_ref, o_ref):
    o_ref[...] = ((p0_ref[0] + p0_ref[1]) + (p1_ref[0] + p1_ref[1])
                  + (p2_ref[0] + p2_ref[1]))


def _fp_sum(parts):
    return pl.pallas_call(
        _fpsum_body,
        out_shape=jax.ShapeDtypeStruct((M_MOL, OUT), jnp.float32),
    )(*parts)


def kernel(atom_repr, bond_repr, atom_nbr_d1, atom_nbr_d2, atom_nbr_d3,
           atom_nbr_d4, bond_nbr_d1, bond_nbr_d2, bond_nbr_d3, bond_nbr_d4,
           mol_ids, W_deg, W_self, conv_bias, W_out, b_out):
    atom_nbrs = (atom_nbr_d1, atom_nbr_d2, atom_nbr_d3, atom_nbr_d4)
    bond_nbrs = (bond_nbr_d1, bond_nbr_d2, bond_nbr_d3, bond_nbr_d4)
    aidx = jnp.concatenate([a.T.reshape(-1) for a in atom_nbrs])
    bidx = jnp.concatenate([b.T.reshape(-1) for b in bond_nbrs])

    b_flat = _bond_gather(bond_repr, bidx)
    bfold = b_flat.reshape(N_ATOM // 8, 1, 8 * EDGE)

    x = atom_repr
    parts = []
    kbs = []
    for l in range(NL):
        wdb = W_deg[l][:, NODE:, :]
        kb = jnp.zeros((4, NODE, 8 * OUT), dtype=jnp.float32)
        for i in range(8):
            kb = kb.at[:, i * EDGE:(i + 1) * EDGE, i * OUT:(i + 1) * OUT].set(wdb)
        kbs.append(kb)

    for l in range(NL):
        a_sum = _conv_gather(x, aidx)
        s, act, st = _layer_tc(x, a_sum, bfold, W_deg[l][:, :NODE, :], kbs[l],
                               W_self[l], W_out[l], b_out[l].reshape(1, OUT),
                               conv_bias[l].reshape(1, OUT))
        parts.append(_segment_scatter(s, mol_ids))
        x = _norm_tc(act, st)
    s = _fp_tc(x, W_out[NL], b_out[NL].reshape(1, OUT))
    parts.append(_segment_scatter(s, mol_ids))
    return _fp_sum(parts)


# speedup vs baseline: 3.3449x; 1.1061x over previous
import jax
import jax.numpy as jnp
from jax import lax
from jax.experimental import pallas as pl
from jax.experimental.pallas import tpu as pltpu
from jax.experimental.pallas import tpu_sc as plsc

N_ATOM = 100000
ND = 25000
NB = 250000
M_MOL = 4096
NODE = 128
EDGE = 16
OUT = 128
NL = 2
DEGS = (1, 2, 3, 4)

NTILE = 32
T_MAIN = 776
C_MAIN = 96
N_CHUNK = 8
T_TAIL = 8
REM_TILES = 21
GOFF = (0, ND, 3 * ND, 6 * ND)


def _make_conv_body(apply_norm):
    def body(*args):
        if apply_norm:
            (x_hbm, aidx_hbm, ns_hbm, a_out,
             idxa0, idxa1, idxa2, idxa3, stga, sacc, nsv,
             gsem0, gsem1, osem0, osem1, tsem) = args
        else:
            (x_hbm, aidx_hbm, a_out,
             idxa0, idxa1, idxa2, idxa3, stga, sacc,
             gsem0, gsem1, osem0, osem1, tsem) = args
            nsv = None
        idxa = (idxa0, idxa1, idxa2, idxa3)
        gsem = (gsem0, gsem1)
        osem = (osem0, osem1)
        cid = lax.axis_index("c")
        sid = lax.axis_index("s")
        w = sid * 2 + cid

        if apply_norm:
            pltpu.sync_copy(ns_hbm, nsv)

        for g, d in enumerate(DEGS):
            goff = GOFF[g]
            orow0 = g * ND
            row_base = orow0 + w * T_MAIN

            for j in range(d):
                pltpu.sync_copy(
                    aidx_hbm.at[pl.ds(goff + j * ND + w * T_MAIN, T_MAIN)],
                    idxa[j].at[pl.ds(0, T_MAIN)])

            def g_issue(b, col0, d=d):
                for j in range(d):
                    pltpu.async_copy(x_hbm.at[idxa[j].at[pl.ds(col0, C_MAIN)]],
                                     stga.at[b, j, :, :], gsem[b])

            def g_wait(b, d=d):
                for j in range(d):
                    pltpu.make_async_copy(
                        x_hbm.at[idxa[j].at[pl.ds(0, C_MAIN)]],
                        stga.at[b, j, :, :], gsem[b]).wait()

            def sum_rows(b, nrow, d=d):
                def srow(r, _):
                    for col in range(NODE // 16):
                        sl = pl.ds(col * 16, 16)
                        if apply_norm:
                            sc = nsv[2, sl]
                            sh = nsv[3, sl]
                            v = jnp.maximum(stga[b, 0, r, sl] * sc + sh, 0.0)
                            for j in range(1, d):
                                v = v + jnp.maximum(
                                    stga[b, j, r, sl] * sc + sh, 0.0)
                        else:
                            v = stga[b, 0, r, sl]
                            for j in range(1, d):
                                v = v + stga[b, j, r, sl]
                        sacc[b, r, sl] = v
                    return 0
                lax.fori_loop(0, nrow, srow, 0, unroll=False)

            def w_issue(b, out_r):
                pltpu.async_copy(sacc.at[b, :, :],
                                 a_out.at[pl.ds(out_r, C_MAIN), :], osem[b])

            def w_wait(b):
                pltpu.make_async_copy(sacc.at[b, :, :],
                                      a_out.at[pl.ds(0, C_MAIN), :],
                                      osem[b]).wait()

            g_issue(0, 0)
            g_issue(1, C_MAIN)

            def lbody(k, _):
                for b in (0, 1):
                    @pl.when(lax.rem(k, 2) == b)
                    def _(b=b):
                        g_wait(b)
                        @pl.when(k >= 2)
                        def _():
                            w_wait(b)
                        sum_rows(b, C_MAIN)
                        w_issue(b, row_base + k * C_MAIN)
                        @pl.when(k + 2 < N_CHUNK)
                        def _():
                            g_issue(b, (k + 2) * C_MAIN)
                return 0
            lax.fori_loop(0, N_CHUNK, lbody, 0, unroll=False)
            w_wait(0)
            w_wait(1)

            def small_chunk(col0, out_r, d=d):
                cps = []
                for j in range(d):
                    cps.append(pltpu.async_copy(
                        x_hbm.at[idxa[j].at[pl.ds(col0, 8)]],
                        stga.at[0, j, pl.ds(0, 8), :], tsem))
                for cp in cps:
                    cp.wait()
                sum_rows(0, 8)
                pltpu.async_copy(sacc.at[0, pl.ds(0, 8), :],
                                 a_out.at[pl.ds(out_r, 8), :], tsem).wait()

            small_chunk(N_CHUNK * C_MAIN, row_base + N_CHUNK * C_MAIN)

            @pl.when(w < REM_TILES)
            def _():
                rbase = NTILE * T_MAIN
                for j in range(d):
                    pltpu.sync_copy(
                        aidx_hbm.at[pl.ds(goff + j * ND + rbase + w * 8, 8)],
                        idxa[j].at[pl.ds(T_MAIN, 8)])
                small_chunk(T_MAIN, orow0 + rbase + w * 8)
    return body


def _conv_gather(x, aidx, ns=None):
    mesh = plsc.VectorSubcoreMesh(core_axis_name="c", subcore_axis_name="s",
                                  num_cores=2, num_subcores=16)
    apply_norm = ns is not None
    scratch = (
        [pltpu.VMEM((T_MAIN + 8,), jnp.int32) for _ in range(4)]
        + [pltpu.VMEM((2, 4, C_MAIN, NODE), jnp.float32),
           pltpu.VMEM((2, C_MAIN, NODE), jnp.float32)]
        + ([pltpu.VMEM((8, NODE), jnp.float32)] if apply_norm else [])
        + [pltpu.SemaphoreType.DMA for _ in range(5)])
    f = pl.kernel(
        _make_conv_body(apply_norm),
        out_type=jax.ShapeDtypeStruct((N_ATOM, NODE), jnp.float32),
        mesh=mesh,
        scratch_types=scratch,
    )
    return f(x, aidx, ns) if apply_norm else f(x, aidx)


def _bond_body(bond_hbm, bidx_hbm, bf_out,
               idxb0, idxb1, idxb2, idxb3, stgb, bacc, gsem, osem):
    idxb = (idxb0, idxb1, idxb2, idxb3)
    cid = lax.axis_index("c")
    sid = lax.axis_index("s")
    w = sid * 2 + cid

    for g, d in enumerate(DEGS):
        goff = GOFF[g]
        orow0 = g * ND

        for j in range(d):
            pltpu.sync_copy(bidx_hbm.at[pl.ds(goff + j * ND + w * T_MAIN, T_MAIN)],
                            idxb[j].at[pl.ds(0, T_MAIN)])

        def chunk(col0, nrow, out_r, d=d):
            cps = []
            for j in range(d):
                cps.append(pltpu.async_copy(
                    bond_hbm.at[idxb[j].at[pl.ds(col0, nrow)]],
                    stgb.at[j, pl.ds(0, nrow), :], gsem))
            for cp in cps:
                cp.wait()

            def srow(r, _):
                v = stgb[0, r, :]
                for j in range(1, d):
                    v = v + stgb[j, r, :]
                bacc[pl.ds(r * EDGE, EDGE)] = v
                return 0
            lax.fori_loop(0, nrow, srow, 0, unroll=False)

            pltpu.async_copy(bacc.at[pl.ds(0, nrow * EDGE)],
                             bf_out.at[pl.ds(out_r * EDGE, nrow * EDGE)],
                             osem).wait()

        row_base = orow0 + w * T_MAIN

        def main(k, _):
            chunk(k * C_MAIN, C_MAIN, row_base + k * C_MAIN)
            return 0
        lax.fori_loop(0, N_CHUNK, main, 0, unroll=False)
        chunk(N_CHUNK * C_MAIN, T_TAIL, row_base + N_CHUNK * C_MAIN)

        @pl.when(w < REM_TILES)
        def _():
            rbase = NTILE * T_MAIN
            for j in range(d):
                pltpu.sync_copy(
                    bidx_hbm.at[pl.ds(goff + j * ND + rbase + w * 8, 8)],
                    idxb[j].at[pl.ds(T_MAIN, 8)])
            chunk(T_MAIN, 8, orow0 + rbase + w * 8)


def _bond_gather(bond, bidx):
    mesh = plsc.VectorSubcoreMesh(core_axis_name="c", subcore_axis_name="s",
                                  num_cores=2, num_subcores=16)
    f = pl.kernel(
        _bond_body,
        out_type=jax.ShapeDtypeStruct((N_ATOM * EDGE,), jnp.float32),
        mesh=mesh,
        scratch_types=(
            [pltpu.VMEM((T_MAIN + 8,), jnp.int32) for _ in range(4)]
            + [
                pltpu.VMEM((4, C_MAIN, EDGE), jnp.float32),
                pltpu.VMEM((C_MAIN * EDGE,), jnp.float32),
                pltpu.SemaphoreType.DMA,
                pltpu.SemaphoreType.DMA,
            ]),
        compiler_params=pltpu.CompilerParams(use_tc_tiling_on_sc=False),
    )
    return f(bond, bidx)


SEG_T = 3128
SEG_C = 128


def _seg_body(s_hbm, ids_hbm, out_hbm, stg, idx0, idx1, idx56, idx8, zbuf,
              acc, lsem0, lsem1):
    idxb = (idx0, idx1)
    lsem = (lsem0, lsem1)
    cid = lax.axis_index("c")
    sid = lax.axis_index("s")
    base = cid * (N_ATOM // 2) + sid * SEG_T

    def ld_issue(b, k):
        r0 = base + k * SEG_C
        pltpu.async_copy(ids_hbm.at[pl.ds(r0, SEG_C)], idxb[b], lsem[b])
        pltpu.async_copy(s_hbm.at[pl.ds(r0, SEG_C), :], stg.at[b], lsem[b])

    def ld_wait(b):
        pltpu.make_async_copy(ids_hbm.at[pl.ds(0, SEG_C)], idxb[b],
                              lsem[b]).wait()
        pltpu.make_async_copy(s_hbm.at[pl.ds(0, SEG_C), :], stg.at[b],
                              lsem[b]).wait()

    ld_issue(0, 0)
    ld_issue(1, 1)

    def zrow(r, _):
        for col in range(NODE // 16):
            zbuf[r, pl.ds(col * 16, 16)] = jnp.zeros((16,), jnp.float32)
        return 0
    lax.fori_loop(0, M_MOL // 16, zrow, 0, unroll=False)
    pltpu.sync_copy(zbuf, acc.at[pl.ds(sid * (M_MOL // 16), M_MOL // 16), :])
    plsc.subcore_barrier()

    def lbody(k, _):
        for b in (0, 1):
            @pl.when(lax.rem(k, 2) == b)
            def _(b=b):
                ld_wait(b)
                pltpu.sync_copy(stg.at[b], acc.at[idxb[b]], add=True)
                @pl.when(k + 2 < 24)
                def _():
                    ld_issue(b, k + 2)
        return 0
    lax.fori_loop(0, 24, lbody, 0, unroll=False)

    r0 = base + 24 * SEG_C

    @pl.when(sid < 15)
    def _():
        pltpu.sync_copy(ids_hbm.at[pl.ds(r0, 56)], idx56)
        pltpu.sync_copy(s_hbm.at[pl.ds(r0, 56), :], stg.at[0, pl.ds(0, 56), :])
        pltpu.sync_copy(stg.at[0, pl.ds(0, 56), :], acc.at[idx56], add=True)

    @pl.when(sid == 15)
    def _():
        pltpu.sync_copy(ids_hbm.at[pl.ds(r0, 8)], idx8)
        pltpu.sync_copy(s_hbm.at[pl.ds(r0, 8), :], stg.at[0, pl.ds(0, 8), :])
        pltpu.sync_copy(stg.at[0, pl.ds(0, 8), :], acc.at[idx8], add=True)

    plsc.subcore_barrier()
    pltpu.sync_copy(acc.at[pl.ds(sid * 256, 256), :],
                    out_hbm.at[cid, pl.ds(sid * 256, 256), :])


def _segment_scatter(s, mol_ids):
    mesh = plsc.VectorSubcoreMesh(core_axis_name="c", subcore_axis_name="s",
                                  num_cores=2, num_subcores=16)
    f = pl.kernel(
        _seg_body,
        out_type=jax.ShapeDtypeStruct((2, M_MOL, OUT), jnp.float32),
        mesh=mesh,
        scratch_types=[
            pltpu.VMEM((2, SEG_C, OUT), jnp.float32),
            pltpu.VMEM((SEG_C,), jnp.int32),
            pltpu.VMEM((SEG_C,), jnp.int32),
            pltpu.VMEM((56,), jnp.int32),
            pltpu.VMEM((8,), jnp.int32),
            pltpu.VMEM((M_MOL // 16, OUT), jnp.float32),
            pltpu.VMEM_SHARED((M_MOL, OUT), jnp.float32),
            pltpu.SemaphoreType.DMA,
            pltpu.SemaphoreType.DMA,
        ],
    )
    return f(s, mol_ids)


BLK = 1000
NBLK = N_ATOM // BLK
BPG = ND // BLK


def _make_layer_body(apply_norm):
    def body(*args):
        if apply_norm:
            (x_ref, ns_ref, a_ref, bf_ref, wda_ref, kb_ref, ws_ref, wo_ref,
             bo_ref, cb_ref, s_ref, act_ref, st_ref) = args
        else:
            (x_ref, a_ref, bf_ref, wda_ref, kb_ref, ws_ref, wo_ref,
             bo_ref, cb_ref, s_ref, act_ref, st_ref) = args
        i = pl.program_id(0)
        x = x_ref[...]
        if apply_norm:
            x = jnp.maximum(x * ns_ref[2][None, :] + ns_ref[3][None, :], 0.0)
        y = (jnp.dot(x, wo_ref[...], preferred_element_type=jnp.float32)
             + bo_ref[0][None, :])
        y = y - jnp.max(y, axis=1, keepdims=True)
        e = jnp.exp(y)
        s_ref[...] = e / jnp.sum(e, axis=1, keepdims=True)

        bf = bf_ref[...].reshape(BLK // 8, NODE)
        bp = jnp.dot(bf, kb_ref[0], preferred_element_type=jnp.float32)
        act = (jnp.dot(x, ws_ref[...], preferred_element_type=jnp.float32)
               + jnp.dot(a_ref[...], wda_ref[0],
                         preferred_element_type=jnp.float32)
               + bp.reshape(BLK, OUT)
               + cb_ref[0][None, :])
        act_ref[...] = act

        @pl.when(i == 0)
        def _():
            st_ref[...] = jnp.zeros_like(st_ref)
        st_ref[0, :] = st_ref[0, :] + jnp.sum(act, axis=0)
        st_ref[1, :] = st_ref[1, :] + jnp.sum(act * act, axis=0)

        @pl.when(i == NBLK - 1)
        def _():
            mean = st_ref[0, :] * (1.0 / N_ATOM)
            var = st_ref[1, :] * (1.0 / N_ATOM) - mean * mean
            inv = lax.rsqrt(var + 1e-5)
            st_ref[2, :] = inv
            st_ref[3, :] = -mean * inv
    return body


def _layer_tc(x, ns, a_sum, bfold, wda, kb, wself, wout, bout, cbias):
    apply_norm = ns is not None
    blockspec_x = pl.BlockSpec((BLK, NODE), lambda i: (i, 0))
    in_specs = [blockspec_x]
    args = [x]
    if apply_norm:
        in_specs.append(pl.BlockSpec((8, NODE), lambda i: (0, 0)))
        args.append(ns)
    in_specs += [
        pl.BlockSpec((BLK, NODE), lambda i: (i, 0)),
        pl.BlockSpec((BLK // 8, 1, NODE), lambda i: (i, 0, 0)),
        pl.BlockSpec((1, NODE, OUT), lambda i: (i // BPG, 0, 0)),
        pl.BlockSpec((1, NODE, 8 * OUT), lambda i: (i // BPG, 0, 0)),
        pl.BlockSpec((NODE, OUT), lambda i: (0, 0)),
        pl.BlockSpec((NODE, OUT), lambda i: (0, 0)),
        pl.BlockSpec((1, OUT), lambda i: (0, 0)),
        pl.BlockSpec((1, OUT), lambda i: (0, 0)),
    ]
    args += [a_sum, bfold, wda, kb, wself, wout, bout, cbias]
    return pl.pallas_call(
        _make_layer_body(apply_norm),
        grid=(NBLK,),
        in_specs=in_specs,
        out_specs=[
            pl.BlockSpec((BLK, OUT), lambda i: (i, 0)),
            pl.BlockSpec((BLK, OUT), lambda i: (i, 0)),
            pl.BlockSpec((8, OUT), lambda i: (0, 0)),
        ],
        out_shape=[
            jax.ShapeDtypeStruct((N_ATOM, OUT), jnp.float32),
            jax.ShapeDtypeStruct((N_ATOM, OUT), jnp.float32),
            jax.ShapeDtypeStruct((8, OUT), jnp.float32),
        ],
        compiler_params=pltpu.CompilerParams(dimension_semantics=("arbitrary",)),
    )(*args)


def _fp_body(x_ref, ns_ref, wo_ref, bo_ref, s_ref):
    x = x_ref[...]
    x = jnp.maximum(x * ns_ref[2][None, :] + ns_ref[3][None, :], 0.0)
    y = (jnp.dot(x, wo_ref[...], preferred_element_type=jnp.float32)
         + bo_ref[0][None, :])
    y = y - jnp.max(y, axis=1, keepdims=True)
    e = jnp.exp(y)
    s_ref[...] = e / jnp.sum(e, axis=1, keepdims=True)


def _fp_tc(x, ns, wout, bout):
    return pl.pallas_call(
        _fp_body,
        grid=(NBLK,),
        in_specs=[
            pl.BlockSpec((BLK, NODE), lambda i: (i, 0)),
            pl.BlockSpec((8, NODE), lambda i: (0, 0)),
            pl.BlockSpec((NODE, OUT), lambda i: (0, 0)),
            pl.BlockSpec((1, OUT), lambda i: (0, 0)),
        ],
        out_specs=pl.BlockSpec((BLK, OUT), lambda i: (i, 0)),
        out_shape=jax.ShapeDtypeStruct((N_ATOM, OUT), jnp.float32),
    )(x, ns, wout, bout)


def _fpsum_body(p0_ref, p1_ref, p2_ref, o_ref):
    o_ref[...] = ((p0_ref[0] + p0_ref[1]) + (p1_ref[0] + p1_ref[1])
                  + (p2_ref[0] + p2_ref[1]))


def _fp_sum(parts):
    return pl.pallas_call(
        _fpsum_body,
        out_shape=jax.ShapeDtypeStruct((M_MOL, OUT), jnp.float32),
    )(*parts)


def kernel(atom_repr, bond_repr, atom_nbr_d1, atom_nbr_d2, atom_nbr_d3,
           atom_nbr_d4, bond_nbr_d1, bond_nbr_d2, bond_nbr_d3, bond_nbr_d4,
           mol_ids, W_deg, W_self, conv_bias, W_out, b_out):
    atom_nbrs = (atom_nbr_d1, atom_nbr_d2, atom_nbr_d3, atom_nbr_d4)
    bond_nbrs = (bond_nbr_d1, bond_nbr_d2, bond_nbr_d3, bond_nbr_d4)
    aidx = jnp.concatenate([a.T.reshape(-1) for a in atom_nbrs])
    bidx = jnp.concatenate([b.T.reshape(-1) for b in bond_nbrs])

    b_flat = _bond_gather(bond_repr, bidx)
    bfold = b_flat.reshape(N_ATOM // 8, 1, 8 * EDGE)

    kbs = []
    for l in range(NL):
        wdb = W_deg[l][:, NODE:, :]
        kb = jnp.zeros((4, NODE, 8 * OUT), dtype=jnp.float32)
        for i in range(8):
            kb = kb.at[:, i * EDGE:(i + 1) * EDGE, i * OUT:(i + 1) * OUT].set(wdb)
        kbs.append(kb)

    x = atom_repr
    ns = None
    parts = []
    for l in range(NL):
        a_sum = _conv_gather(x, aidx, ns)
        s, act, st = _layer_tc(x, ns, a_sum, bfold, W_deg[l][:, :NODE, :],
                               kbs[l], W_self[l], W_out[l],
                               b_out[l].reshape(1, OUT),
                               conv_bias[l].reshape(1, OUT))
        parts.append(_segment_scatter(s, mol_ids))
        x = act
        ns = st
    s = _fp_tc(x, ns, W_out[NL], b_out[NL].reshape(1, OUT))
    parts.append(_segment_scatter(s, mol_ids))
    return _fp_sum(parts)


# speedup vs baseline: 3.9738x; 1.1880x over previous
import jax
import jax.numpy as jnp
from jax import lax
from jax.experimental import pallas as pl
from jax.experimental.pallas import tpu as pltpu
from jax.experimental.pallas import tpu_sc as plsc

N_ATOM = 100000
ND = 25000
NB = 250000
M_MOL = 4096
NODE = 128
EDGE = 16
OUT = 128
NL = 2
DEGS = (1, 2, 3, 4)

NTILE = 32
T_MAIN = 776
C_MAIN = 96
N_CHUNK = 8
T_TAIL = 8
REM_TILES = 21
GOFF = (0, ND, 3 * ND, 6 * ND)


def _make_conv_body(apply_norm):
    def body(*args):
        if apply_norm:
            (x_hbm, aidx_hbm, ns_hbm, a_out,
             idxa0, idxa1, idxa2, idxa3, stga, tstg, nsv,
             gsem0, gsem1, osem0, osem1, tsem, rsem) = args
        else:
            (x_hbm, aidx_hbm, a_out,
             idxa0, idxa1, idxa2, idxa3, stga, tstg,
             gsem0, gsem1, osem0, osem1, tsem, rsem) = args
            nsv = None
        idxa = (idxa0, idxa1, idxa2, idxa3)
        gsem = (gsem0, gsem1)
        osem = (osem0, osem1)
        cid = lax.axis_index("c")
        sid = lax.axis_index("s")
        w = sid * 2 + cid

        if apply_norm:
            pltpu.sync_copy(ns_hbm, nsv)
            scs = [nsv[2, pl.ds(c * 16, 16)] for c in range(NODE // 16)]
            shs = [nsv[3, pl.ds(c * 16, 16)] for c in range(NODE // 16)]
        else:
            scs = shs = None

        for g, d in enumerate(DEGS):
            goff = GOFF[g]
            orow0 = g * ND
            row_base = orow0 + w * T_MAIN
            rbase = NTILE * T_MAIN

            for j in range(d):
                pltpu.sync_copy(
                    aidx_hbm.at[pl.ds(goff + j * ND + w * T_MAIN, T_MAIN)],
                    idxa[j].at[pl.ds(0, T_MAIN)])

            @pl.when(w < REM_TILES)
            def _():
                for j in range(d):
                    pltpu.sync_copy(
                        aidx_hbm.at[pl.ds(goff + j * ND + rbase + w * 8, 8)],
                        idxa[j].at[pl.ds(T_MAIN, 8)])

            def g_issue(b, col0, d=d):
                for j in range(d):
                    pltpu.async_copy(x_hbm.at[idxa[j].at[pl.ds(col0, C_MAIN)]],
                                     stga.at[b, j, :, :], gsem[b])

            def g_wait(b, d=d):
                for j in range(d):
                    pltpu.make_async_copy(
                        x_hbm.at[idxa[j].at[pl.ds(0, C_MAIN)]],
                        stga.at[b, j, :, :], gsem[b]).wait()

            def sum_rows(stg, nrow, d=d):
                def srow(r, _):
                    for col in range(NODE // 16):
                        sl = pl.ds(col * 16, 16)
                        if apply_norm:
                            v = jnp.maximum(
                                stg[0, r, sl] * scs[col] + shs[col], 0.0)
                            for j in range(1, d):
                                v = v + jnp.maximum(
                                    stg[j, r, sl] * scs[col] + shs[col], 0.0)
                        else:
                            v = stg[0, r, sl]
                            for j in range(1, d):
                                v = v + stg[j, r, sl]
                        stg[0, r, sl] = v
                    return 0
                lax.fori_loop(0, nrow, srow, 0, unroll=False)

            def w_wait(b):
                pltpu.make_async_copy(stga.at[b, 0, :, :],
                                      a_out.at[pl.ds(0, C_MAIN), :],
                                      osem[b]).wait()

            tail_cps = []
            for j in range(d):
                tail_cps.append(pltpu.async_copy(
                    x_hbm.at[idxa[j].at[pl.ds(N_CHUNK * C_MAIN, 8)]],
                    tstg.at[j, pl.ds(0, 8), :], tsem))

            @pl.when(w < REM_TILES)
            def _():
                for j in range(d):
                    pltpu.async_copy(
                        x_hbm.at[idxa[j].at[pl.ds(T_MAIN, 8)]],
                        tstg.at[j, pl.ds(8, 8), :], rsem)

            g_issue(0, 0)

            def lbody(k, _):
                for b in (0, 1):
                    @pl.when(lax.rem(k, 2) == b)
                    def _(b=b):
                        bo = 1 - b
                        @pl.when(k >= 1)
                        def _():
                            w_wait(bo)
                        @pl.when(k + 1 < N_CHUNK)
                        def _():
                            g_issue(bo, (k + 1) * C_MAIN)
                        g_wait(b)
                        sum_rows(stga.at[b], C_MAIN)
                        pltpu.async_copy(
                            stga.at[b, 0, :, :],
                            a_out.at[pl.ds(row_base + k * C_MAIN, C_MAIN), :],
                            osem[b])
                return 0
            lax.fori_loop(0, N_CHUNK, lbody, 0, unroll=False)
            w_wait((N_CHUNK - 1) % 2)

            for cp in tail_cps:
                cp.wait()
            sum_rows(tstg, 8)
            pltpu.async_copy(tstg.at[0, pl.ds(0, 8), :],
                             a_out.at[pl.ds(row_base + N_CHUNK * C_MAIN, 8), :],
                             tsem).wait()

            @pl.when(w < REM_TILES)
            def _():
                for j in range(d):
                    pltpu.make_async_copy(
                        x_hbm.at[idxa[j].at[pl.ds(T_MAIN, 8)]],
                        tstg.at[j, pl.ds(8, 8), :], rsem).wait()
                def srow(r, _):
                    for col in range(NODE // 16):
                        sl = pl.ds(col * 16, 16)
                        if apply_norm:
                            v = jnp.maximum(
                                tstg[0, r, sl] * scs[col] + shs[col], 0.0)
                            for j in range(1, d):
                                v = v + jnp.maximum(
                                    tstg[j, r, sl] * scs[col] + shs[col], 0.0)
                        else:
                            v = tstg[0, r, sl]
                            for j in range(1, d):
                                v = v + tstg[j, r, sl]
                        tstg[0, r, sl] = v
                    return 0
                lax.fori_loop(8, 16, srow, 0, unroll=False)
                pltpu.async_copy(tstg.at[0, pl.ds(8, 8), :],
                                 a_out.at[pl.ds(orow0 + rbase + w * 8, 8), :],
                                 tsem).wait()
    return body


def _conv_gather(x, aidx, ns=None):
    mesh = plsc.VectorSubcoreMesh(core_axis_name="c", subcore_axis_name="s",
                                  num_cores=2, num_subcores=16)
    apply_norm = ns is not None
    scratch = (
        [pltpu.VMEM((T_MAIN + 8,), jnp.int32) for _ in range(4)]
        + [pltpu.VMEM((2, 4, C_MAIN, NODE), jnp.float32),
           pltpu.VMEM((4, 16, NODE), jnp.float32)]
        + ([pltpu.VMEM((8, NODE), jnp.float32)] if apply_norm else [])
        + [pltpu.SemaphoreType.DMA for _ in range(6)])
    f = pl.kernel(
        _make_conv_body(apply_norm),
        out_type=jax.ShapeDtypeStruct((N_ATOM, NODE), jnp.float32),
        mesh=mesh,
        scratch_types=scratch,
    )
    return f(x, aidx, ns) if apply_norm else f(x, aidx)


def _bond_body(bond_hbm, bidx_hbm, bf_out,
               idxb0, idxb1, idxb2, idxb3, stgb, bacc, gsem, osem):
    idxb = (idxb0, idxb1, idxb2, idxb3)
    cid = lax.axis_index("c")
    sid = lax.axis_index("s")
    w = sid * 2 + cid

    for g, d in enumerate(DEGS):
        goff = GOFF[g]
        orow0 = g * ND

        for j in range(d):
            pltpu.sync_copy(bidx_hbm.at[pl.ds(goff + j * ND + w * T_MAIN, T_MAIN)],
                            idxb[j].at[pl.ds(0, T_MAIN)])

        def chunk(col0, nrow, out_r, d=d):
            cps = []
            for j in range(d):
                cps.append(pltpu.async_copy(
                    bond_hbm.at[idxb[j].at[pl.ds(col0, nrow)]],
                    stgb.at[j, pl.ds(0, nrow), :], gsem))
            for cp in cps:
                cp.wait()

            def srow(r, _):
                v = stgb[0, r, :]
                for j in range(1, d):
                    v = v + stgb[j, r, :]
                bacc[pl.ds(r * EDGE, EDGE)] = v
                return 0
            lax.fori_loop(0, nrow, srow, 0, unroll=False)

            pltpu.async_copy(bacc.at[pl.ds(0, nrow * EDGE)],
                             bf_out.at[pl.ds(out_r * EDGE, nrow * EDGE)],
                             osem).wait()

        row_base = orow0 + w * T_MAIN

        def main(k, _):
            chunk(k * C_MAIN, C_MAIN, row_base + k * C_MAIN)
            return 0
        lax.fori_loop(0, N_CHUNK, main, 0, unroll=False)
        chunk(N_CHUNK * C_MAIN, T_TAIL, row_base + N_CHUNK * C_MAIN)

        @pl.when(w < REM_TILES)
        def _():
            rbase = NTILE * T_MAIN
            for j in range(d):
                pltpu.sync_copy(
                    bidx_hbm.at[pl.ds(goff + j * ND + rbase + w * 8, 8)],
                    idxb[j].at[pl.ds(T_MAIN, 8)])
            chunk(T_MAIN, 8, orow0 + rbase + w * 8)


def _bond_gather(bond, bidx):
    mesh = plsc.VectorSubcoreMesh(core_axis_name="c", subcore_axis_name="s",
                                  num_cores=2, num_subcores=16)
    f = pl.kernel(
        _bond_body,
        out_type=jax.ShapeDtypeStruct((N_ATOM * EDGE,), jnp.float32),
        mesh=mesh,
        scratch_types=(
            [pltpu.VMEM((T_MAIN + 8,), jnp.int32) for _ in range(4)]
            + [
                pltpu.VMEM((4, C_MAIN, EDGE), jnp.float32),
                pltpu.VMEM((C_MAIN * EDGE,), jnp.float32),
                pltpu.SemaphoreType.DMA,
                pltpu.SemaphoreType.DMA,
            ]),
        compiler_params=pltpu.CompilerParams(use_tc_tiling_on_sc=False),
    )
    return f(bond, bidx)


SEG_T = 3128
SEG_C = 128


def _seg_body(s_hbm, ids_hbm, out_hbm, stg, idx0, idx1, idx56, idx8, zbuf,
              acc, lsem0, lsem1):
    idxb = (idx0, idx1)
    lsem = (lsem0, lsem1)
    cid = lax.axis_index("c")
    sid = lax.axis_index("s")
    base = cid * (N_ATOM // 2) + sid * SEG_T

    def ld_issue(b, k):
        r0 = base + k * SEG_C
        pltpu.async_copy(ids_hbm.at[pl.ds(r0, SEG_C)], idxb[b], lsem[b])
        pltpu.async_copy(s_hbm.at[pl.ds(r0, SEG_C), :], stg.at[b], lsem[b])

    def ld_wait(b):
        pltpu.make_async_copy(ids_hbm.at[pl.ds(0, SEG_C)], idxb[b],
                              lsem[b]).wait()
        pltpu.make_async_copy(s_hbm.at[pl.ds(0, SEG_C), :], stg.at[b],
                              lsem[b]).wait()

    ld_issue(0, 0)
    ld_issue(1, 1)

    def zrow(r, _):
        for col in range(NODE // 16):
            zbuf[r, pl.ds(col * 16, 16)] = jnp.zeros((16,), jnp.float32)
        return 0
    lax.fori_loop(0, M_MOL // 16, zrow, 0, unroll=False)
    pltpu.sync_copy(zbuf, acc.at[pl.ds(sid * (M_MOL // 16), M_MOL // 16), :])
    plsc.subcore_barrier()

    def lbody(k, _):
        for b in (0, 1):
            @pl.when(lax.rem(k, 2) == b)
            def _(b=b):
                ld_wait(b)
                pltpu.sync_copy(stg.at[b], acc.at[idxb[b]], add=True)
                @pl.when(k + 2 < 24)
                def _():
                    ld_issue(b, k + 2)
        return 0
    lax.fori_loop(0, 24, lbody, 0, unroll=False)

    r0 = base + 24 * SEG_C

    @pl.when(sid < 15)
    def _():
        pltpu.sync_copy(ids_hbm.at[pl.ds(r0, 56)], idx56)
        pltpu.sync_copy(s_hbm.at[pl.ds(r0, 56), :], stg.at[0, pl.ds(0, 56), :])
        pltpu.sync_copy(stg.at[0, pl.ds(0, 56), :], acc.at[idx56], add=True)

    @pl.when(sid == 15)
    def _():
        pltpu.sync_copy(ids_hbm.at[pl.ds(r0, 8)], idx8)
        pltpu.sync_copy(s_hbm.at[pl.ds(r0, 8), :], stg.at[0, pl.ds(0, 8), :])
        pltpu.sync_copy(stg.at[0, pl.ds(0, 8), :], acc.at[idx8], add=True)

    plsc.subcore_barrier()
    pltpu.sync_copy(acc.at[pl.ds(sid * 256, 256), :],
                    out_hbm.at[cid, pl.ds(sid * 256, 256), :])


def _segment_scatter(s, mol_ids):
    mesh = plsc.VectorSubcoreMesh(core_axis_name="c", subcore_axis_name="s",
                                  num_cores=2, num_subcores=16)
    f = pl.kernel(
        _seg_body,
        out_type=jax.ShapeDtypeStruct((2, M_MOL, OUT), jnp.float32),
        mesh=mesh,
        scratch_types=[
            pltpu.VMEM((2, SEG_C, OUT), jnp.float32),
            pltpu.VMEM((SEG_C,), jnp.int32),
            pltpu.VMEM((SEG_C,), jnp.int32),
            pltpu.VMEM((56,), jnp.int32),
            pltpu.VMEM((8,), jnp.int32),
            pltpu.VMEM((M_MOL // 16, OUT), jnp.float32),
            pltpu.VMEM_SHARED((M_MOL, OUT), jnp.float32),
            pltpu.SemaphoreType.DMA,
            pltpu.SemaphoreType.DMA,
        ],
    )
    return f(s, mol_ids)


BLK = 1000
NBLK = N_ATOM // BLK
BPG = ND // BLK


def _make_layer_body(apply_norm):
    def body(*args):
        if apply_norm:
            (x_ref, ns_ref, a_ref, bf_ref, wda_ref, kb_ref, ws_ref, wo_ref,
             bo_ref, cb_ref, s_ref, act_ref, st_ref) = args
        else:
            (x_ref, a_ref, bf_ref, wda_ref, kb_ref, ws_ref, wo_ref,
             bo_ref, cb_ref, s_ref, act_ref, st_ref) = args
        i = pl.program_id(0)
        x = x_ref[...]
        if apply_norm:
            x = jnp.maximum(x * ns_ref[2][None, :] + ns_ref[3][None, :], 0.0)
        y = (jnp.dot(x, wo_ref[...], preferred_element_type=jnp.float32)
             + bo_ref[0][None, :])
        y = y - jnp.max(y, axis=1, keepdims=True)
        e = jnp.exp(y)
        s_ref[...] = e / jnp.sum(e, axis=1, keepdims=True)

        bf = bf_ref[...].reshape(BLK // 8, NODE)
        bp = jnp.dot(bf, kb_ref[0], preferred_element_type=jnp.float32)
        act = (jnp.dot(x, ws_ref[...], preferred_element_type=jnp.float32)
               + jnp.dot(a_ref[...], wda_ref[0],
                         preferred_element_type=jnp.float32)
               + bp.reshape(BLK, OUT)
               + cb_ref[0][None, :])
        act_ref[...] = act

        @pl.when(i == 0)
        def _():
            st_ref[...] = jnp.zeros_like(st_ref)
        st_ref[0, :] = st_ref[0, :] + jnp.sum(act, axis=0)
        st_ref[1, :] = st_ref[1, :] + jnp.sum(act * act, axis=0)

        @pl.when(i == NBLK - 1)
        def _():
            mean = st_ref[0, :] * (1.0 / N_ATOM)
            var = st_ref[1, :] * (1.0 / N_ATOM) - mean * mean
            inv = lax.rsqrt(var + 1e-5)
            st_ref[2, :] = inv
            st_ref[3, :] = -mean * inv
    return body


def _layer_tc(x, ns, a_sum, bfold, wda, kb, wself, wout, bout, cbias):
    apply_norm = ns is not None
    blockspec_x = pl.BlockSpec((BLK, NODE), lambda i: (i, 0))
    in_specs = [blockspec_x]
    args = [x]
    if apply_norm:
        in_specs.append(pl.BlockSpec((8, NODE), lambda i: (0, 0)))
        args.append(ns)
    in_specs += [
        pl.BlockSpec((BLK, NODE), lambda i: (i, 0)),
        pl.BlockSpec((BLK // 8, 1, NODE), lambda i: (i, 0, 0)),
        pl.BlockSpec((1, NODE, OUT), lambda i: (i // BPG, 0, 0)),
        pl.BlockSpec((1, NODE, 8 * OUT), lambda i: (i // BPG, 0, 0)),
        pl.BlockSpec((NODE, OUT), lambda i: (0, 0)),
        pl.BlockSpec((NODE, OUT), lambda i: (0, 0)),
        pl.BlockSpec((1, OUT), lambda i: (0, 0)),
        pl.BlockSpec((1, OUT), lambda i: (0, 0)),
    ]
    args += [a_sum, bfold, wda, kb, wself, wout, bout, cbias]
    return pl.pallas_call(
        _make_layer_body(apply_norm),
        grid=(NBLK,),
        in_specs=in_specs,
        out_specs=[
            pl.BlockSpec((BLK, OUT), lambda i: (i, 0)),
            pl.BlockSpec((BLK, OUT), lambda i: (i, 0)),
            pl.BlockSpec((8, OUT), lambda i: (0, 0)),
        ],
        out_shape=[
            jax.ShapeDtypeStruct((N_ATOM, OUT), jnp.float32),
            jax.ShapeDtypeStruct((N_ATOM, OUT), jnp.float32),
            jax.ShapeDtypeStruct((8, OUT), jnp.float32),
        ],
        compiler_params=pltpu.CompilerParams(dimension_semantics=("arbitrary",)),
    )(*args)


def _fp_body(x_ref, ns_ref, wo_ref, bo_ref, s_ref):
    x = x_ref[...]
    x = jnp.maximum(x * ns_ref[2][None, :] + ns_ref[3][None, :], 0.0)
    y = (jnp.dot(x, wo_ref[...], preferred_element_type=jnp.float32)
         + bo_ref[0][None, :])
    y = y - jnp.max(y, axis=1, keepdims=True)
    e = jnp.exp(y)
    s_ref[...] = e / jnp.sum(e, axis=1, keepdims=True)


def _fp_tc(x, ns, wout, bout):
    return pl.pallas_call(
        _fp_body,
        grid=(NBLK,),
        in_specs=[
            pl.BlockSpec((BLK, NODE), lambda i: (i, 0)),
            pl.BlockSpec((8, NODE), lambda i: (0, 0)),
            pl.BlockSpec((NODE, OUT), lambda i: (0, 0)),
            pl.BlockSpec((1, OUT), lambda i: (0, 0)),
        ],
        out_specs=pl.BlockSpec((BLK, OUT), lambda i: (i, 0)),
        out_shape=jax.ShapeDtypeStruct((N_ATOM, OUT), jnp.float32),
    )(x, ns, wout, bout)


def _fpsum_body(p0_ref, p1_ref, p2_ref, o_ref):
    o_ref[...] = ((p0_ref[0] + p0_ref[1]) + (p1_ref[0] + p1_ref[1])
                  + (p2_ref[0] + p2_ref[1]))


def _fp_sum(parts):
    return pl.pallas_call(
        _fpsum_body,
        out_shape=jax.ShapeDtypeStruct((M_MOL, OUT), jnp.float32),
    )(*parts)


def kernel(atom_repr, bond_repr, atom_nbr_d1, atom_nbr_d2, atom_nbr_d3,
           atom_nbr_d4, bond_nbr_d1, bond_nbr_d2, bond_nbr_d3, bond_nbr_d4,
           mol_ids, W_deg, W_self, conv_bias, W_out, b_out):
    atom_nbrs = (atom_nbr_d1, atom_nbr_d2, atom_nbr_d3, atom_nbr_d4)
    bond_nbrs = (bond_nbr_d1, bond_nbr_d2, bond_nbr_d3, bond_nbr_d4)
    aidx = jnp.concatenate([a.T.reshape(-1) for a in atom_nbrs])
    bidx = jnp.concatenate([b.T.reshape(-1) for b in bond_nbrs])

    b_flat = _bond_gather(bond_repr, bidx)
    bfold = b_flat.reshape(N_ATOM // 8, 1, 8 * EDGE)

    kbs = []
    for l in range(NL):
        wdb = W_deg[l][:, NODE:, :]
        kb = jnp.zeros((4, NODE, 8 * OUT), dtype=jnp.float32)
        for i in range(8):
            kb = kb.at[:, i * EDGE:(i + 1) * EDGE, i * OUT:(i + 1) * OUT].set(wdb)
        kbs.append(kb)

    x = atom_repr
    ns = None
    parts = []
    for l in range(NL):
        a_sum = _conv_gather(x, aidx, ns)
        s, act, st = _layer_tc(x, ns, a_sum, bfold, W_deg[l][:, :NODE, :],
                               kbs[l], W_self[l], W_out[l],
                               b_out[l].reshape(1, OUT),
                               conv_bias[l].reshape(1, OUT))
        parts.append(_segment_scatter(s, mol_ids))
        x = act
        ns = st
    s = _fp_tc(x, ns, W_out[NL], b_out[NL].reshape(1, OUT))
    parts.append(_segment_scatter(s, mol_ids))
    return _fp_sum(parts)


# speedup vs baseline: 4.8495x; 1.2204x over previous
import jax
import jax.numpy as jnp
from jax import lax
from jax.experimental import pallas as pl
from jax.experimental.pallas import tpu as pltpu
from jax.experimental.pallas import tpu_sc as plsc

N_ATOM = 100000
ND = 25000
NB = 250000
M_MOL = 4096
NODE = 128
EDGE = 16
OUT = 128
NL = 2
DEGS = (1, 2, 3, 4)

NTILE = 32
T_MAIN = 776
C_MAIN = 96
N_CHUNK = 8
T_TAIL = 8
REM_TILES = 21
GOFF = (0, ND, 3 * ND, 6 * ND)


def _make_conv_body(apply_norm):
    def body(*args):
        if apply_norm:
            (x_hbm, aidx_hbm, ns_hbm, a_out,
             idxa0, idxa1, idxa2, idxa3, stga, tstg, nsv,
             gsem0, gsem1, osem0, osem1, tsem, rsem) = args
        else:
            (x_hbm, aidx_hbm, a_out,
             idxa0, idxa1, idxa2, idxa3, stga, tstg,
             gsem0, gsem1, osem0, osem1, tsem, rsem) = args
            nsv = None
        idxa = (idxa0, idxa1, idxa2, idxa3)
        gsem = (gsem0, gsem1)
        osem = (osem0, osem1)
        cid = lax.axis_index("c")
        sid = lax.axis_index("s")
        w = sid * 2 + cid

        if apply_norm:
            pltpu.sync_copy(ns_hbm, nsv)
            scs = [nsv[2, pl.ds(c * 16, 16)] for c in range(NODE // 16)]
            shs = [nsv[3, pl.ds(c * 16, 16)] for c in range(NODE // 16)]
        else:
            scs = shs = None

        for g, d in enumerate(DEGS):
            goff = GOFF[g]
            orow0 = g * ND
            row_base = orow0 + w * T_MAIN
            rbase = NTILE * T_MAIN

            for j in range(d):
                pltpu.sync_copy(
                    aidx_hbm.at[pl.ds(goff + j * ND + w * T_MAIN, T_MAIN)],
                    idxa[j].at[pl.ds(0, T_MAIN)])

            @pl.when(w < REM_TILES)
            def _():
                for j in range(d):
                    pltpu.sync_copy(
                        aidx_hbm.at[pl.ds(goff + j * ND + rbase + w * 8, 8)],
                        idxa[j].at[pl.ds(T_MAIN, 8)])

            def g_issue(b, col0, d=d):
                for j in range(d):
                    pltpu.async_copy(x_hbm.at[idxa[j].at[pl.ds(col0, C_MAIN)]],
                                     stga.at[b, j, :, :], gsem[b])

            def g_wait(b, d=d):
                for j in range(d):
                    pltpu.make_async_copy(
                        x_hbm.at[idxa[j].at[pl.ds(0, C_MAIN)]],
                        stga.at[b, j, :, :], gsem[b]).wait()

            def sum_rows(stg, nrow, d=d):
                def srow(r2, _):
                    r = r2 * 2
                    for rr in (r, r + 1):
                        for col in range(NODE // 16):
                            sl = pl.ds(col * 16, 16)
                            if apply_norm:
                                v = jnp.maximum(
                                    stg[0, rr, sl] * scs[col] + shs[col], 0.0)
                                for j in range(1, d):
                                    v = v + jnp.maximum(
                                        stg[j, rr, sl] * scs[col] + shs[col],
                                        0.0)
                            else:
                                v = stg[0, rr, sl]
                                for j in range(1, d):
                                    v = v + stg[j, rr, sl]
                            stg[0, rr, sl] = v
                    return 0
                lax.fori_loop(0, nrow // 2, srow, 0, unroll=False)

            def w_wait(b):
                pltpu.make_async_copy(stga.at[b, 0, :, :],
                                      a_out.at[pl.ds(0, C_MAIN), :],
                                      osem[b]).wait()

            tail_cps = []
            for j in range(d):
                tail_cps.append(pltpu.async_copy(
                    x_hbm.at[idxa[j].at[pl.ds(N_CHUNK * C_MAIN, 8)]],
                    tstg.at[j, pl.ds(0, 8), :], tsem))

            @pl.when(w < REM_TILES)
            def _():
                for j in range(d):
                    pltpu.async_copy(
                        x_hbm.at[idxa[j].at[pl.ds(T_MAIN, 8)]],
                        tstg.at[j, pl.ds(8, 8), :], rsem)

            g_issue(0, 0)

            def lbody(k, _):
                for b in (0, 1):
                    @pl.when(lax.rem(k, 2) == b)
                    def _(b=b):
                        bo = 1 - b
                        @pl.when(k >= 1)
                        def _():
                            w_wait(bo)
                        @pl.when(k + 1 < N_CHUNK)
                        def _():
                            g_issue(bo, (k + 1) * C_MAIN)
                        g_wait(b)
                        sum_rows(stga.at[b], C_MAIN)
                        pltpu.async_copy(
                            stga.at[b, 0, :, :],
                            a_out.at[pl.ds(row_base + k * C_MAIN, C_MAIN), :],
                            osem[b])
                return 0
            lax.fori_loop(0, N_CHUNK, lbody, 0, unroll=False)
            w_wait((N_CHUNK - 1) % 2)

            for cp in tail_cps:
                cp.wait()
            sum_rows(tstg, 8)
            pltpu.async_copy(tstg.at[0, pl.ds(0, 8), :],
                             a_out.at[pl.ds(row_base + N_CHUNK * C_MAIN, 8), :],
                             tsem).wait()

            @pl.when(w < REM_TILES)
            def _():
                for j in range(d):
                    pltpu.make_async_copy(
                        x_hbm.at[idxa[j].at[pl.ds(T_MAIN, 8)]],
                        tstg.at[j, pl.ds(8, 8), :], rsem).wait()
                def srow(r, _):
                    for col in range(NODE // 16):
                        sl = pl.ds(col * 16, 16)
                        if apply_norm:
                            v = jnp.maximum(
                                tstg[0, r, sl] * scs[col] + shs[col], 0.0)
                            for j in range(1, d):
                                v = v + jnp.maximum(
                                    tstg[j, r, sl] * scs[col] + shs[col], 0.0)
                        else:
                            v = tstg[0, r, sl]
                            for j in range(1, d):
                                v = v + tstg[j, r, sl]
                        tstg[0, r, sl] = v
                    return 0
                lax.fori_loop(8, 16, srow, 0, unroll=False)
                pltpu.async_copy(tstg.at[0, pl.ds(8, 8), :],
                                 a_out.at[pl.ds(orow0 + rbase + w * 8, 8), :],
                                 tsem).wait()
    return body


def _conv_gather(x, aidx, ns=None):
    mesh = plsc.VectorSubcoreMesh(core_axis_name="c", subcore_axis_name="s",
                                  num_cores=2, num_subcores=16)
    apply_norm = ns is not None
    scratch = (
        [pltpu.VMEM((T_MAIN + 8,), jnp.int32) for _ in range(4)]
        + [pltpu.VMEM((2, 4, C_MAIN, NODE), jnp.float32),
           pltpu.VMEM((4, 16, NODE), jnp.float32)]
        + ([pltpu.VMEM((8, NODE), jnp.float32)] if apply_norm else [])
        + [pltpu.SemaphoreType.DMA for _ in range(6)])
    f = pl.kernel(
        _make_conv_body(apply_norm),
        out_type=jax.ShapeDtypeStruct((N_ATOM, NODE), jnp.float32),
        mesh=mesh,
        scratch_types=scratch,
    )
    return f(x, aidx, ns) if apply_norm else f(x, aidx)


def _bond_body(bond_hbm, bidx_hbm, bf_out,
               idxb0, idxb1, idxb2, idxb3, stgb, tstgb, bacc,
               gsem0, gsem1, osem0, osem1, tsem, rsem):
    idxb = (idxb0, idxb1, idxb2, idxb3)
    gsem = (gsem0, gsem1)
    osem = (osem0, osem1)
    cid = lax.axis_index("c")
    sid = lax.axis_index("s")
    w = sid * 2 + cid

    for g, d in enumerate(DEGS):
        goff = GOFF[g]
        orow0 = g * ND
        row_base = orow0 + w * T_MAIN
        rbase = NTILE * T_MAIN

        for j in range(d):
            pltpu.sync_copy(bidx_hbm.at[pl.ds(goff + j * ND + w * T_MAIN, T_MAIN)],
                            idxb[j].at[pl.ds(0, T_MAIN)])

        @pl.when(w < REM_TILES)
        def _():
            for j in range(d):
                pltpu.sync_copy(
                    bidx_hbm.at[pl.ds(goff + j * ND + rbase + w * 8, 8)],
                    idxb[j].at[pl.ds(T_MAIN, 8)])

        def g_issue(b, col0, d=d):
            for j in range(d):
                pltpu.async_copy(bond_hbm.at[idxb[j].at[pl.ds(col0, C_MAIN)]],
                                 stgb.at[b, j, :, :], gsem[b])

        def g_wait(b, d=d):
            for j in range(d):
                pltpu.make_async_copy(
                    bond_hbm.at[idxb[j].at[pl.ds(0, C_MAIN)]],
                    stgb.at[b, j, :, :], gsem[b]).wait()

        def sum_rows(stg, bac, nrow, d=d):
            def srow(r2, _):
                r = r2 * 2
                for rr in (r, r + 1):
                    v = stg[0, rr, :]
                    for j in range(1, d):
                        v = v + stg[j, rr, :]
                    bac[pl.ds(rr * EDGE, EDGE)] = v
                return 0
            lax.fori_loop(0, nrow // 2, srow, 0, unroll=False)

        def w_wait(b):
            pltpu.make_async_copy(bacc.at[b, pl.ds(0, C_MAIN * EDGE)],
                                  bf_out.at[pl.ds(0, C_MAIN * EDGE)],
                                  osem[b]).wait()

        tail_cps = []
        for j in range(d):
            tail_cps.append(pltpu.async_copy(
                bond_hbm.at[idxb[j].at[pl.ds(N_CHUNK * C_MAIN, 8)]],
                tstgb.at[j, pl.ds(0, 8), :], tsem))

        @pl.when(w < REM_TILES)
        def _():
            for j in range(d):
                pltpu.async_copy(
                    bond_hbm.at[idxb[j].at[pl.ds(T_MAIN, 8)]],
                    tstgb.at[j, pl.ds(8, 8), :], rsem)

        g_issue(0, 0)

        def lbody(k, _):
            for b in (0, 1):
                @pl.when(lax.rem(k, 2) == b)
                def _(b=b):
                    bo = 1 - b
                    @pl.when(k >= 1)
                    def _():
                        w_wait(bo)
                    @pl.when(k + 1 < N_CHUNK)
                    def _():
                        g_issue(bo, (k + 1) * C_MAIN)
                    g_wait(b)
                    sum_rows(stgb.at[b], bacc.at[b], C_MAIN)
                    pltpu.async_copy(
                        bacc.at[b, pl.ds(0, C_MAIN * EDGE)],
                        bf_out.at[pl.ds((row_base + k * C_MAIN) * EDGE,
                                        C_MAIN * EDGE)], osem[b])
            return 0
        lax.fori_loop(0, N_CHUNK, lbody, 0, unroll=False)
        w_wait((N_CHUNK - 1) % 2)

        for cp in tail_cps:
            cp.wait()
        sum_rows(tstgb, bacc.at[0], 8)
        pltpu.async_copy(bacc.at[0, pl.ds(0, 8 * EDGE)],
                         bf_out.at[pl.ds((row_base + N_CHUNK * C_MAIN) * EDGE,
                                         8 * EDGE)], tsem).wait()

        @pl.when(w < REM_TILES)
        def _():
            for j in range(d):
                pltpu.make_async_copy(
                    bond_hbm.at[idxb[j].at[pl.ds(T_MAIN, 8)]],
                    tstgb.at[j, pl.ds(8, 8), :], rsem).wait()
            def srow(r2, _):
                r = 8 + r2 * 2
                for rr in (r, r + 1):
                    v = tstgb[0, rr, :]
                    for j in range(1, d):
                        v = v + tstgb[j, rr, :]
                    bacc[0, pl.ds(rr * EDGE, EDGE)] = v
                return 0
            lax.fori_loop(0, 4, srow, 0, unroll=False)
            pltpu.async_copy(bacc.at[0, pl.ds(8 * EDGE, 8 * EDGE)],
                             bf_out.at[pl.ds((orow0 + rbase + w * 8) * EDGE,
                                             8 * EDGE)], tsem).wait()


def _bond_gather(bond, bidx):
    mesh = plsc.VectorSubcoreMesh(core_axis_name="c", subcore_axis_name="s",
                                  num_cores=2, num_subcores=16)
    f = pl.kernel(
        _bond_body,
        out_type=jax.ShapeDtypeStruct((N_ATOM * EDGE,), jnp.float32),
        mesh=mesh,
        scratch_types=(
            [pltpu.VMEM((T_MAIN + 8,), jnp.int32) for _ in range(4)]
            + [
                pltpu.VMEM((2, 4, C_MAIN, EDGE), jnp.float32),
                pltpu.VMEM((4, 16, EDGE), jnp.float32),
                pltpu.VMEM((2, C_MAIN * EDGE), jnp.float32),
            ]
            + [pltpu.SemaphoreType.DMA for _ in range(6)]),
        compiler_params=pltpu.CompilerParams(use_tc_tiling_on_sc=False),
    )
    return f(bond, bidx)


SEG_T = 3128
SEG_C = 128


def _seg_body(s_hbm, ids_hbm, out_hbm, stg, idx0, idx1, idx56, idx8, zbuf,
              acc, lsem0, lsem1):
    idxb = (idx0, idx1)
    lsem = (lsem0, lsem1)
    cid = lax.axis_index("c")
    sid = lax.axis_index("s")
    base = cid * (N_ATOM // 2) + sid * SEG_T

    def ld_issue(b, k):
        r0 = base + k * SEG_C
        pltpu.async_copy(ids_hbm.at[pl.ds(r0, SEG_C)], idxb[b], lsem[b])
        pltpu.async_copy(s_hbm.at[pl.ds(r0, SEG_C), :], stg.at[b], lsem[b])

    def ld_wait(b):
        pltpu.make_async_copy(ids_hbm.at[pl.ds(0, SEG_C)], idxb[b],
                              lsem[b]).wait()
        pltpu.make_async_copy(s_hbm.at[pl.ds(0, SEG_C), :], stg.at[b],
                              lsem[b]).wait()

    ld_issue(0, 0)
    ld_issue(1, 1)

    def zrow(r, _):
        for col in range(NODE // 16):
            zbuf[r, pl.ds(col * 16, 16)] = jnp.zeros((16,), jnp.float32)
        return 0
    lax.fori_loop(0, M_MOL // 16, zrow, 0, unroll=False)
    pltpu.sync_copy(zbuf, acc.at[pl.ds(sid * (M_MOL // 16), M_MOL // 16), :])
    plsc.subcore_barrier()

    def lbody(k, _):
        for b in (0, 1):
            @pl.when(lax.rem(k, 2) == b)
            def _(b=b):
                ld_wait(b)
                pltpu.sync_copy(stg.at[b], acc.at[idxb[b]], add=True)
                @pl.when(k + 2 < 24)
                def _():
                    ld_issue(b, k + 2)
        return 0
    lax.fori_loop(0, 24, lbody, 0, unroll=False)

    r0 = base + 24 * SEG_C

    @pl.when(sid < 15)
    def _():
        pltpu.sync_copy(ids_hbm.at[pl.ds(r0, 56)], idx56)
        pltpu.sync_copy(s_hbm.at[pl.ds(r0, 56), :], stg.at[0, pl.ds(0, 56), :])
        pltpu.sync_copy(stg.at[0, pl.ds(0, 56), :], acc.at[idx56], add=True)

    @pl.when(sid == 15)
    def _():
        pltpu.sync_copy(ids_hbm.at[pl.ds(r0, 8)], idx8)
        pltpu.sync_copy(s_hbm.at[pl.ds(r0, 8), :], stg.at[0, pl.ds(0, 8), :])
        pltpu.sync_copy(stg.at[0, pl.ds(0, 8), :], acc.at[idx8], add=True)

    plsc.subcore_barrier()
    pltpu.sync_copy(acc.at[pl.ds(sid * 256, 256), :],
                    out_hbm.at[cid, pl.ds(sid * 256, 256), :])


def _segment_scatter(s, mol_ids):
    mesh = plsc.VectorSubcoreMesh(core_axis_name="c", subcore_axis_name="s",
                                  num_cores=2, num_subcores=16)
    f = pl.kernel(
        _seg_body,
        out_type=jax.ShapeDtypeStruct((2, M_MOL, OUT), jnp.float32),
        mesh=mesh,
        scratch_types=[
            pltpu.VMEM((2, SEG_C, OUT), jnp.float32),
            pltpu.VMEM((SEG_C,), jnp.int32),
            pltpu.VMEM((SEG_C,), jnp.int32),
            pltpu.VMEM((56,), jnp.int32),
            pltpu.VMEM((8,), jnp.int32),
            pltpu.VMEM((M_MOL // 16, OUT), jnp.float32),
            pltpu.VMEM_SHARED((M_MOL, OUT), jnp.float32),
            pltpu.SemaphoreType.DMA,
            pltpu.SemaphoreType.DMA,
        ],
    )
    return f(s, mol_ids)


BLK = 5000
NBLK = N_ATOM // BLK
BPG = ND // BLK


def _make_layer_body(apply_norm):
    def body(*args):
        if apply_norm:
            (x_ref, ns_ref, a_ref, bf_ref, wda_ref, kb_ref, ws_ref, wo_ref,
             bo_ref, cb_ref, s_ref, act_ref, st_ref) = args
        else:
            (x_ref, a_ref, bf_ref, wda_ref, kb_ref, ws_ref, wo_ref,
             bo_ref, cb_ref, s_ref, act_ref, st_ref) = args
        i = pl.program_id(0)
        x = x_ref[...]
        if apply_norm:
            x = jnp.maximum(x * ns_ref[2][None, :] + ns_ref[3][None, :], 0.0)
        y = (jnp.dot(x, wo_ref[...], preferred_element_type=jnp.float32)
             + bo_ref[0][None, :])
        y = y - jnp.max(y, axis=1, keepdims=True)
        e = jnp.exp(y)
        s_ref[...] = e / jnp.sum(e, axis=1, keepdims=True)

        bf = bf_ref[...].reshape(BLK // 8, NODE)
        bp = jnp.dot(bf, kb_ref[0], preferred_element_type=jnp.float32)
        act = (jnp.dot(x, ws_ref[...], preferred_element_type=jnp.float32)
               + jnp.dot(a_ref[...], wda_ref[0],
                         preferred_element_type=jnp.float32)
               + bp.reshape(BLK, OUT)
               + cb_ref[0][None, :])
        act_ref[...] = act

        @pl.when(i == 0)
        def _():
            st_ref[...] = jnp.zeros_like(st_ref)
        st_ref[0, :] = st_ref[0, :] + jnp.sum(act, axis=0)
        st_ref[1, :] = st_ref[1, :] + jnp.sum(act * act, axis=0)

        @pl.when(i == NBLK - 1)
        def _():
            mean = st_ref[0, :] * (1.0 / N_ATOM)
            var = st_ref[1, :] * (1.0 / N_ATOM) - mean * mean
            inv = lax.rsqrt(var + 1e-5)
            st_ref[2, :] = inv
            st_ref[3, :] = -mean * inv
    return body


def _layer_tc(x, ns, a_sum, bfold, wda, kb, wself, wout, bout, cbias):
    apply_norm = ns is not None
    blockspec_x = pl.BlockSpec((BLK, NODE), lambda i: (i, 0))
    in_specs = [blockspec_x]
    args = [x]
    if apply_norm:
        in_specs.append(pl.BlockSpec((8, NODE), lambda i: (0, 0)))
        args.append(ns)
    in_specs += [
        pl.BlockSpec((BLK, NODE), lambda i: (i, 0)),
        pl.BlockSpec((BLK // 8, 1, NODE), lambda i: (i, 0, 0)),
        pl.BlockSpec((1, NODE, OUT), lambda i: (i // BPG, 0, 0)),
        pl.BlockSpec((1, NODE, 8 * OUT), lambda i: (i // BPG, 0, 0)),
        pl.BlockSpec((NODE, OUT), lambda i: (0, 0)),
        pl.BlockSpec((NODE, OUT), lambda i: (0, 0)),
        pl.BlockSpec((1, OUT), lambda i: (0, 0)),
        pl.BlockSpec((1, OUT), lambda i: (0, 0)),
    ]
    args += [a_sum, bfold, wda, kb, wself, wout, bout, cbias]
    return pl.pallas_call(
        _make_layer_body(apply_norm),
        grid=(NBLK,),
        in_specs=in_specs,
        out_specs=[
            pl.BlockSpec((BLK, OUT), lambda i: (i, 0)),
            pl.BlockSpec((BLK, OUT), lambda i: (i, 0)),
            pl.BlockSpec((8, OUT), lambda i: (0, 0)),
        ],
        out_shape=[
            jax.ShapeDtypeStruct((N_ATOM, OUT), jnp.float32),
            jax.ShapeDtypeStruct((N_ATOM, OUT), jnp.float32),
            jax.ShapeDtypeStruct((8, OUT), jnp.float32),
        ],
        compiler_params=pltpu.CompilerParams(dimension_semantics=("arbitrary",)),
    )(*args)


def _fp_body(x_ref, ns_ref, wo_ref, bo_ref, s_ref):
    x = x_ref[...]
    x = jnp.maximum(x * ns_ref[2][None, :] + ns_ref[3][None, :], 0.0)
    y = (jnp.dot(x, wo_ref[...], preferred_element_type=jnp.float32)
         + bo_ref[0][None, :])
    y = y - jnp.max(y, axis=1, keepdims=True)
    e = jnp.exp(y)
    s_ref[...] = e / jnp.sum(e, axis=1, keepdims=True)


def _fp_tc(x, ns, wout, bout):
    return pl.pallas_call(
        _fp_body,
        grid=(NBLK,),
        in_specs=[
            pl.BlockSpec((BLK, NODE), lambda i: (i, 0)),
            pl.BlockSpec((8, NODE), lambda i: (0, 0)),
            pl.BlockSpec((NODE, OUT), lambda i: (0, 0)),
            pl.BlockSpec((1, OUT), lambda i: (0, 0)),
        ],
        out_specs=pl.BlockSpec((BLK, OUT), lambda i: (i, 0)),
        out_shape=jax.ShapeDtypeStruct((N_ATOM, OUT), jnp.float32),
    )(x, ns, wout, bout)


def _fpsum_body(p0_ref, p1_ref, p2_ref, o_ref):
    o_ref[...] = ((p0_ref[0] + p0_ref[1]) + (p1_ref[0] + p1_ref[1])
                  + (p2_ref[0] + p2_ref[1]))


def _fp_sum(parts):
    return pl.pallas_call(
        _fpsum_body,
        out_shape=jax.ShapeDtypeStruct((M_MOL, OUT), jnp.float32),
    )(*parts)


def kernel(atom_repr, bond_repr, atom_nbr_d1, atom_nbr_d2, atom_nbr_d3,
           atom_nbr_d4, bond_nbr_d1, bond_nbr_d2, bond_nbr_d3, bond_nbr_d4,
           mol_ids, W_deg, W_self, conv_bias, W_out, b_out):
    atom_nbrs = (atom_nbr_d1, atom_nbr_d2, atom_nbr_d3, atom_nbr_d4)
    bond_nbrs = (bond_nbr_d1, bond_nbr_d2, bond_nbr_d3, bond_nbr_d4)
    aidx = jnp.concatenate([a.T.reshape(-1) for a in atom_nbrs])
    bidx = jnp.concatenate([b.T.reshape(-1) for b in bond_nbrs])

    b_flat = _bond_gather(bond_repr, bidx)
    bfold = b_flat.reshape(N_ATOM // 8, 1, 8 * EDGE)

    kbs = []
    for l in range(NL):
        wdb = W_deg[l][:, NODE:, :]
        kb = jnp.zeros((4, NODE, 8 * OUT), dtype=jnp.float32)
        for i in range(8):
            kb = kb.at[:, i * EDGE:(i + 1) * EDGE, i * OUT:(i + 1) * OUT].set(wdb)
        kbs.append(kb)

    x = atom_repr
    ns = None
    parts = []
    for l in range(NL):
        a_sum = _conv_gather(x, aidx, ns)
        s, act, st = _layer_tc(x, ns, a_sum, bfold, W_deg[l][:, :NODE, :],
                               kbs[l], W_self[l], W_out[l],
                               b_out[l].reshape(1, OUT),
                               conv_bias[l].reshape(1, OUT))
        parts.append(_segment_scatter(s, mol_ids))
        x = act
        ns = st
    s = _fp_tc(x, ns, W_out[NL], b_out[NL].reshape(1, OUT))
    parts.append(_segment_scatter(s, mol_ids))
    return _fp_sum(parts)


# speedup vs baseline: 4.9062x; 1.0117x over previous
import jax
import jax.numpy as jnp
from jax import lax
from jax.experimental import pallas as pl
from jax.experimental.pallas import tpu as pltpu
from jax.experimental.pallas import tpu_sc as plsc

N_ATOM = 100000
ND = 25000
NB = 250000
M_MOL = 4096
NODE = 128
EDGE = 16
OUT = 128
NL = 2
DEGS = (1, 2, 3, 4)

NTILE = 32
T_MAIN = 776
C_MAIN = 96
N_CHUNK = 8
T_TAIL = 8
REM_TILES = 21
GOFF = (0, ND, 3 * ND, 6 * ND)


def _make_conv_body(apply_norm):
    def body(*args):
        if apply_norm:
            (x_hbm, aidx_hbm, ns_hbm, a_out,
             idxa0, idxa1, idxa2, idxa3, stga, tstg, nsv,
             gsem0, gsem1, osem0, osem1, tsem, rsem) = args
        else:
            (x_hbm, aidx_hbm, a_out,
             idxa0, idxa1, idxa2, idxa3, stga, tstg,
             gsem0, gsem1, osem0, osem1, tsem, rsem) = args
            nsv = None
        idxa = (idxa0, idxa1, idxa2, idxa3)
        gsem = (gsem0, gsem1)
        osem = (osem0, osem1)
        cid = lax.axis_index("c")
        sid = lax.axis_index("s")
        w = sid * 2 + cid

        if apply_norm:
            pltpu.sync_copy(ns_hbm, nsv)
            scs = [nsv[2, pl.ds(c * 16, 16)] for c in range(NODE // 16)]
            shs = [nsv[3, pl.ds(c * 16, 16)] for c in range(NODE // 16)]
        else:
            scs = shs = None

        for g, d in enumerate(DEGS):
            goff = GOFF[g]
            orow0 = g * ND
            row_base = orow0 + w * T_MAIN
            rbase = NTILE * T_MAIN

            for j in range(d):
                pltpu.sync_copy(
                    aidx_hbm.at[pl.ds(goff + j * ND + w * T_MAIN, T_MAIN)],
                    idxa[j].at[pl.ds(0, T_MAIN)])

            @pl.when(w < REM_TILES)
            def _():
                for j in range(d):
                    pltpu.sync_copy(
                        aidx_hbm.at[pl.ds(goff + j * ND + rbase + w * 8, 8)],
                        idxa[j].at[pl.ds(T_MAIN, 8)])

            def g_issue(b, col0, d=d):
                for j in range(d):
                    pltpu.async_copy(x_hbm.at[idxa[j].at[pl.ds(col0, C_MAIN)]],
                                     stga.at[b, j, :, :], gsem[b])

            def g_wait(b, d=d):
                for j in range(d):
                    pltpu.make_async_copy(
                        x_hbm.at[idxa[j].at[pl.ds(0, C_MAIN)]],
                        stga.at[b, j, :, :], gsem[b]).wait()

            def sum_rows(stg, nrow, d=d):
                @plsc.parallel_loop(0, nrow, step=2)
                def srow(r):
                    for rr in (r, r + 1):
                        for col in range(NODE // 16):
                            sl = pl.ds(col * 16, 16)
                            if apply_norm:
                                v = jnp.maximum(
                                    stg[0, rr, sl] * scs[col] + shs[col], 0.0)
                                for j in range(1, d):
                                    v = v + jnp.maximum(
                                        stg[j, rr, sl] * scs[col] + shs[col],
                                        0.0)
                            else:
                                v = stg[0, rr, sl]
                                for j in range(1, d):
                                    v = v + stg[j, rr, sl]
                            stg[0, rr, sl] = v

            def w_wait(b):
                pltpu.make_async_copy(stga.at[b, 0, :, :],
                                      a_out.at[pl.ds(0, C_MAIN), :],
                                      osem[b]).wait()

            tail_cps = []
            for j in range(d):
                tail_cps.append(pltpu.async_copy(
                    x_hbm.at[idxa[j].at[pl.ds(N_CHUNK * C_MAIN, 8)]],
                    tstg.at[j, pl.ds(0, 8), :], tsem))

            @pl.when(w < REM_TILES)
            def _():
                for j in range(d):
                    pltpu.async_copy(
                        x_hbm.at[idxa[j].at[pl.ds(T_MAIN, 8)]],
                        tstg.at[j, pl.ds(8, 8), :], rsem)

            g_issue(0, 0)

            def lbody(k, _):
                for b in (0, 1):
                    @pl.when(lax.rem(k, 2) == b)
                    def _(b=b):
                        bo = 1 - b
                        @pl.when(k >= 1)
                        def _():
                            w_wait(bo)
                        @pl.when(k + 1 < N_CHUNK)
                        def _():
                            g_issue(bo, (k + 1) * C_MAIN)
                        g_wait(b)
                        sum_rows(stga.at[b], C_MAIN)
                        pltpu.async_copy(
                            stga.at[b, 0, :, :],
                            a_out.at[pl.ds(row_base + k * C_MAIN, C_MAIN), :],
                            osem[b])
                return 0
            lax.fori_loop(0, N_CHUNK, lbody, 0, unroll=False)
            w_wait((N_CHUNK - 1) % 2)

            for cp in tail_cps:
                cp.wait()
            sum_rows(tstg, 8)
            pltpu.async_copy(tstg.at[0, pl.ds(0, 8), :],
                             a_out.at[pl.ds(row_base + N_CHUNK * C_MAIN, 8), :],
                             tsem).wait()

            @pl.when(w < REM_TILES)
            def _():
                for j in range(d):
                    pltpu.make_async_copy(
                        x_hbm.at[idxa[j].at[pl.ds(T_MAIN, 8)]],
                        tstg.at[j, pl.ds(8, 8), :], rsem).wait()
                @plsc.parallel_loop(8, 16)
                def srow(r):
                    for col in range(NODE // 16):
                        sl = pl.ds(col * 16, 16)
                        if apply_norm:
                            v = jnp.maximum(
                                tstg[0, r, sl] * scs[col] + shs[col], 0.0)
                            for j in range(1, d):
                                v = v + jnp.maximum(
                                    tstg[j, r, sl] * scs[col] + shs[col], 0.0)
                        else:
                            v = tstg[0, r, sl]
                            for j in range(1, d):
                                v = v + tstg[j, r, sl]
                        tstg[0, r, sl] = v
                pltpu.async_copy(tstg.at[0, pl.ds(8, 8), :],
                                 a_out.at[pl.ds(orow0 + rbase + w * 8, 8), :],
                                 tsem).wait()
    return body


def _conv_gather(x, aidx, ns=None):
    mesh = plsc.VectorSubcoreMesh(core_axis_name="c", subcore_axis_name="s",
                                  num_cores=2, num_subcores=16)
    apply_norm = ns is not None
    scratch = (
        [pltpu.VMEM((T_MAIN + 8,), jnp.int32) for _ in range(4)]
        + [pltpu.VMEM((2, 4, C_MAIN, NODE), jnp.float32),
           pltpu.VMEM((4, 16, NODE), jnp.float32)]
        + ([pltpu.VMEM((8, NODE), jnp.float32)] if apply_norm else [])
        + [pltpu.SemaphoreType.DMA for _ in range(6)])
    f = pl.kernel(
        _make_conv_body(apply_norm),
        out_type=jax.ShapeDtypeStruct((N_ATOM, NODE), jnp.float32),
        mesh=mesh,
        scratch_types=scratch,
    )
    return f(x, aidx, ns) if apply_norm else f(x, aidx)


def _bond_body(bond_hbm, bidx_hbm, bf_out,
               idxb0, idxb1, idxb2, idxb3, stgb, tstgb, bacc,
               gsem0, gsem1, osem0, osem1, tsem, rsem):
    idxb = (idxb0, idxb1, idxb2, idxb3)
    gsem = (gsem0, gsem1)
    osem = (osem0, osem1)
    cid = lax.axis_index("c")
    sid = lax.axis_index("s")
    w = sid * 2 + cid

    for g, d in enumerate(DEGS):
        goff = GOFF[g]
        orow0 = g * ND
        row_base = orow0 + w * T_MAIN
        rbase = NTILE * T_MAIN

        for j in range(d):
            pltpu.sync_copy(bidx_hbm.at[pl.ds(goff + j * ND + w * T_MAIN, T_MAIN)],
                            idxb[j].at[pl.ds(0, T_MAIN)])

        @pl.when(w < REM_TILES)
        def _():
            for j in range(d):
                pltpu.sync_copy(
                    bidx_hbm.at[pl.ds(goff + j * ND + rbase + w * 8, 8)],
                    idxb[j].at[pl.ds(T_MAIN, 8)])

        def g_issue(b, col0, d=d):
            for j in range(d):
                pltpu.async_copy(bond_hbm.at[idxb[j].at[pl.ds(col0, C_MAIN)]],
                                 stgb.at[b, j, :, :], gsem[b])

        def g_wait(b, d=d):
            for j in range(d):
                pltpu.make_async_copy(
                    bond_hbm.at[idxb[j].at[pl.ds(0, C_MAIN)]],
                    stgb.at[b, j, :, :], gsem[b]).wait()

        def sum_rows(stg, bac, nrow, d=d):
            @plsc.parallel_loop(0, nrow, step=2)
            def srow(r):
                for rr in (r, r + 1):
                    v = stg[0, rr, :]
                    for j in range(1, d):
                        v = v + stg[j, rr, :]
                    bac[pl.ds(rr * EDGE, EDGE)] = v

        def w_wait(b):
            pltpu.make_async_copy(bacc.at[b, pl.ds(0, C_MAIN * EDGE)],
                                  bf_out.at[pl.ds(0, C_MAIN * EDGE)],
                                  osem[b]).wait()

        tail_cps = []
        for j in range(d):
            tail_cps.append(pltpu.async_copy(
                bond_hbm.at[idxb[j].at[pl.ds(N_CHUNK * C_MAIN, 8)]],
                tstgb.at[j, pl.ds(0, 8), :], tsem))

        @pl.when(w < REM_TILES)
        def _():
            for j in range(d):
                pltpu.async_copy(
                    bond_hbm.at[idxb[j].at[pl.ds(T_MAIN, 8)]],
                    tstgb.at[j, pl.ds(8, 8), :], rsem)

        g_issue(0, 0)

        def lbody(k, _):
            for b in (0, 1):
                @pl.when(lax.rem(k, 2) == b)
                def _(b=b):
                    bo = 1 - b
                    @pl.when(k >= 1)
                    def _():
                        w_wait(bo)
                    @pl.when(k + 1 < N_CHUNK)
                    def _():
                        g_issue(bo, (k + 1) * C_MAIN)
                    g_wait(b)
                    sum_rows(stgb.at[b], bacc.at[b], C_MAIN)
                    pltpu.async_copy(
                        bacc.at[b, pl.ds(0, C_MAIN * EDGE)],
                        bf_out.at[pl.ds((row_base + k * C_MAIN) * EDGE,
                                        C_MAIN * EDGE)], osem[b])
            return 0
        lax.fori_loop(0, N_CHUNK, lbody, 0, unroll=False)
        w_wait((N_CHUNK - 1) % 2)

        for cp in tail_cps:
            cp.wait()
        sum_rows(tstgb, bacc.at[0], 8)
        pltpu.async_copy(bacc.at[0, pl.ds(0, 8 * EDGE)],
                         bf_out.at[pl.ds((row_base + N_CHUNK * C_MAIN) * EDGE,
                                         8 * EDGE)], tsem).wait()

        @pl.when(w < REM_TILES)
        def _():
            for j in range(d):
                pltpu.make_async_copy(
                    bond_hbm.at[idxb[j].at[pl.ds(T_MAIN, 8)]],
                    tstgb.at[j, pl.ds(8, 8), :], rsem).wait()
            @plsc.parallel_loop(8, 16, step=2)
            def srow(r):
                for rr in (r, r + 1):
                    v = tstgb[0, rr, :]
                    for j in range(1, d):
                        v = v + tstgb[j, rr, :]
                    bacc[0, pl.ds(rr * EDGE, EDGE)] = v
            pltpu.async_copy(bacc.at[0, pl.ds(8 * EDGE, 8 * EDGE)],
                             bf_out.at[pl.ds((orow0 + rbase + w * 8) * EDGE,
                                             8 * EDGE)], tsem).wait()


def _bond_gather(bond, bidx):
    mesh = plsc.VectorSubcoreMesh(core_axis_name="c", subcore_axis_name="s",
                                  num_cores=2, num_subcores=16)
    f = pl.kernel(
        _bond_body,
        out_type=jax.ShapeDtypeStruct((N_ATOM * EDGE,), jnp.float32),
        mesh=mesh,
        scratch_types=(
            [pltpu.VMEM((T_MAIN + 8,), jnp.int32) for _ in range(4)]
            + [
                pltpu.VMEM((2, 4, C_MAIN, EDGE), jnp.float32),
                pltpu.VMEM((4, 16, EDGE), jnp.float32),
                pltpu.VMEM((2, C_MAIN * EDGE), jnp.float32),
            ]
            + [pltpu.SemaphoreType.DMA for _ in range(6)]),
        compiler_params=pltpu.CompilerParams(use_tc_tiling_on_sc=False),
    )
    return f(bond, bidx)


SEG_T = 3128
SEG_C = 128


def _seg_body(s_hbm, ids_hbm, out_hbm, stg, idx0, idx1, idx56, idx8, zbuf,
              acc, lsem0, lsem1):
    idxb = (idx0, idx1)
    lsem = (lsem0, lsem1)
    cid = lax.axis_index("c")
    sid = lax.axis_index("s")
    base = cid * (N_ATOM // 2) + sid * SEG_T

    def ld_issue(b, k):
        r0 = base + k * SEG_C
        pltpu.async_copy(ids_hbm.at[pl.ds(r0, SEG_C)], idxb[b], lsem[b])
        pltpu.async_copy(s_hbm.at[pl.ds(r0, SEG_C), :], stg.at[b], lsem[b])

    def ld_wait(b):
        pltpu.make_async_copy(ids_hbm.at[pl.ds(0, SEG_C)], idxb[b],
                              lsem[b]).wait()
        pltpu.make_async_copy(s_hbm.at[pl.ds(0, SEG_C), :], stg.at[b],
                              lsem[b]).wait()

    ld_issue(0, 0)
    ld_issue(1, 1)

    @plsc.parallel_loop(0, M_MOL // 16)
    def zrow(r):
        for col in range(NODE // 16):
            zbuf[r, pl.ds(col * 16, 16)] = jnp.zeros((16,), jnp.float32)
    pltpu.sync_copy(zbuf, acc.at[pl.ds(sid * (M_MOL // 16), M_MOL // 16), :])
    plsc.subcore_barrier()

    def lbody(k, _):
        for b in (0, 1):
            @pl.when(lax.rem(k, 2) == b)
            def _(b=b):
                ld_wait(b)
                pltpu.sync_copy(stg.at[b], acc.at[idxb[b]], add=True)
                @pl.when(k + 2 < 24)
                def _():
                    ld_issue(b, k + 2)
        return 0
    lax.fori_loop(0, 24, lbody, 0, unroll=False)

    r0 = base + 24 * SEG_C

    @pl.when(sid < 15)
    def _():
        pltpu.sync_copy(ids_hbm.at[pl.ds(r0, 56)], idx56)
        pltpu.sync_copy(s_hbm.at[pl.ds(r0, 56), :], stg.at[0, pl.ds(0, 56), :])
        pltpu.sync_copy(stg.at[0, pl.ds(0, 56), :], acc.at[idx56], add=True)

    @pl.when(sid == 15)
    def _():
        pltpu.sync_copy(ids_hbm.at[pl.ds(r0, 8)], idx8)
        pltpu.sync_copy(s_hbm.at[pl.ds(r0, 8), :], stg.at[0, pl.ds(0, 8), :])
        pltpu.sync_copy(stg.at[0, pl.ds(0, 8), :], acc.at[idx8], add=True)

    plsc.subcore_barrier()
    pltpu.sync_copy(acc.at[pl.ds(sid * 256, 256), :],
                    out_hbm.at[cid, pl.ds(sid * 256, 256), :])


def _segment_scatter(s, mol_ids):
    mesh = plsc.VectorSubcoreMesh(core_axis_name="c", subcore_axis_name="s",
                                  num_cores=2, num_subcores=16)
    f = pl.kernel(
        _seg_body,
        out_type=jax.ShapeDtypeStruct((2, M_MOL, OUT), jnp.float32),
        mesh=mesh,
        scratch_types=[
            pltpu.VMEM((2, SEG_C, OUT), jnp.float32),
            pltpu.VMEM((SEG_C,), jnp.int32),
            pltpu.VMEM((SEG_C,), jnp.int32),
            pltpu.VMEM((56,), jnp.int32),
            pltpu.VMEM((8,), jnp.int32),
            pltpu.VMEM((M_MOL // 16, OUT), jnp.float32),
            pltpu.VMEM_SHARED((M_MOL, OUT), jnp.float32),
            pltpu.SemaphoreType.DMA,
            pltpu.SemaphoreType.DMA,
        ],
    )
    return f(s, mol_ids)


BLK = 5000
NBLK = N_ATOM // BLK
BPG = ND // BLK


def _make_layer_body(apply_norm):
    def body(*args):
        if apply_norm:
            (x_ref, ns_ref, a_ref, bf_ref, wda_ref, kb_ref, ws_ref, wo_ref,
             bo_ref, cb_ref, s_ref, act_ref, st_ref) = args
        else:
            (x_ref, a_ref, bf_ref, wda_ref, kb_ref, ws_ref, wo_ref,
             bo_ref, cb_ref, s_ref, act_ref, st_ref) = args
        i = pl.program_id(0)
        x = x_ref[...]
        if apply_norm:
            x = jnp.maximum(x * ns_ref[2][None, :] + ns_ref[3][None, :], 0.0)
        y = (jnp.dot(x, wo_ref[...], preferred_element_type=jnp.float32)
             + bo_ref[0][None, :])
        y = y - jnp.max(y, axis=1, keepdims=True)
        e = jnp.exp(y)
        s_ref[...] = e / jnp.sum(e, axis=1, keepdims=True)

        bf = bf_ref[...].reshape(BLK // 8, NODE)
        bp = jnp.dot(bf, kb_ref[0], preferred_element_type=jnp.float32)
        act = (jnp.dot(x, ws_ref[...], preferred_element_type=jnp.float32)
               + jnp.dot(a_ref[...], wda_ref[0],
                         preferred_element_type=jnp.float32)
               + bp.reshape(BLK, OUT)
               + cb_ref[0][None, :])
        act_ref[...] = act

        @pl.when(i == 0)
        def _():
            st_ref[...] = jnp.zeros_like(st_ref)
        st_ref[0, :] = st_ref[0, :] + jnp.sum(act, axis=0)
        st_ref[1, :] = st_ref[1, :] + jnp.sum(act * act, axis=0)

        @pl.when(i == NBLK - 1)
        def _():
            mean = st_ref[0, :] * (1.0 / N_ATOM)
            var = st_ref[1, :] * (1.0 / N_ATOM) - mean * mean
            inv = lax.rsqrt(var + 1e-5)
            st_ref[2, :] = inv
            st_ref[3, :] = -mean * inv
    return body


def _layer_tc(x, ns, a_sum, bfold, wda, kb, wself, wout, bout, cbias):
    apply_norm = ns is not None
    blockspec_x = pl.BlockSpec((BLK, NODE), lambda i: (i, 0))
    in_specs = [blockspec_x]
    args = [x]
    if apply_norm:
        in_specs.append(pl.BlockSpec((8, NODE), lambda i: (0, 0)))
        args.append(ns)
    in_specs += [
        pl.BlockSpec((BLK, NODE), lambda i: (i, 0)),
        pl.BlockSpec((BLK // 8, 1, NODE), lambda i: (i, 0, 0)),
        pl.BlockSpec((1, NODE, OUT), lambda i: (i // BPG, 0, 0)),
        pl.BlockSpec((1, NODE, 8 * OUT), lambda i: (i // BPG, 0, 0)),
        pl.BlockSpec((NODE, OUT), lambda i: (0, 0)),
        pl.BlockSpec((NODE, OUT), lambda i: (0, 0)),
        pl.BlockSpec((1, OUT), lambda i: (0, 0)),
        pl.BlockSpec((1, OUT), lambda i: (0, 0)),
    ]
    args += [a_sum, bfold, wda, kb, wself, wout, bout, cbias]
    return pl.pallas_call(
        _make_layer_body(apply_norm),
        grid=(NBLK,),
        in_specs=in_specs,
        out_specs=[
            pl.BlockSpec((BLK, OUT), lambda i: (i, 0)),
            pl.BlockSpec((BLK, OUT), lambda i: (i, 0)),
            pl.BlockSpec((8, OUT), lambda i: (0, 0)),
        ],
        out_shape=[
            jax.ShapeDtypeStruct((N_ATOM, OUT), jnp.float32),
            jax.ShapeDtypeStruct((N_ATOM, OUT), jnp.float32),
            jax.ShapeDtypeStruct((8, OUT), jnp.float32),
        ],
        compiler_params=pltpu.CompilerParams(dimension_semantics=("arbitrary",)),
    )(*args)


def _fp_body(x_ref, ns_ref, wo_ref, bo_ref, s_ref):
    x = x_ref[...]
    x = jnp.maximum(x * ns_ref[2][None, :] + ns_ref[3][None, :], 0.0)
    y = (jnp.dot(x, wo_ref[...], preferred_element_type=jnp.float32)
         + bo_ref[0][None, :])
    y = y - jnp.max(y, axis=1, keepdims=True)
    e = jnp.exp(y)
    s_ref[...] = e / jnp.sum(e, axis=1, keepdims=True)


def _fp_tc(x, ns, wout, bout):
    return pl.pallas_call(
        _fp_body,
        grid=(NBLK,),
        in_specs=[
            pl.BlockSpec((BLK, NODE), lambda i: (i, 0)),
            pl.BlockSpec((8, NODE), lambda i: (0, 0)),
            pl.BlockSpec((NODE, OUT), lambda i: (0, 0)),
            pl.BlockSpec((1, OUT), lambda i: (0, 0)),
        ],
        out_specs=pl.BlockSpec((BLK, OUT), lambda i: (i, 0)),
        out_shape=jax.ShapeDtypeStruct((N_ATOM, OUT), jnp.float32),
    )(x, ns, wout, bout)


def _fpsum_body(p0_ref, p1_ref, p2_ref, o_ref):
    o_ref[...] = ((p0_ref[0] + p0_ref[1]) + (p1_ref[0] + p1_ref[1])
                  + (p2_ref[0] + p2_ref[1]))


def _fp_sum(parts):
    return pl.pallas_call(
        _fpsum_body,
        out_shape=jax.ShapeDtypeStruct((M_MOL, OUT), jnp.float32),
    )(*parts)


def kernel(atom_repr, bond_repr, atom_nbr_d1, atom_nbr_d2, atom_nbr_d3,
           atom_nbr_d4, bond_nbr_d1, bond_nbr_d2, bond_nbr_d3, bond_nbr_d4,
           mol_ids, W_deg, W_self, conv_bias, W_out, b_out):
    atom_nbrs = (atom_nbr_d1, atom_nbr_d2, atom_nbr_d3, atom_nbr_d4)
    bond_nbrs = (bond_nbr_d1, bond_nbr_d2, bond_nbr_d3, bond_nbr_d4)
    aidx = jnp.concatenate([a.T.reshape(-1) for a in atom_nbrs])
    bidx = jnp.concatenate([b.T.reshape(-1) for b in bond_nbrs])

    b_flat = _bond_gather(bond_repr, bidx)
    bfold = b_flat.reshape(N_ATOM // 8, 1, 8 * EDGE)

    kbs = []
    for l in range(NL):
        wdb = W_deg[l][:, NODE:, :]
        kb = jnp.zeros((4, NODE, 8 * OUT), dtype=jnp.float32)
        for i in range(8):
            kb = kb.at[:, i * EDGE:(i + 1) * EDGE, i * OUT:(i + 1) * OUT].set(wdb)
        kbs.append(kb)

    x = atom_repr
    ns = None
    parts = []
    for l in range(NL):
        a_sum = _conv_gather(x, aidx, ns)
        s, act, st = _layer_tc(x, ns, a_sum, bfold, W_deg[l][:, :NODE, :],
                               kbs[l], W_self[l], W_out[l],
                               b_out[l].reshape(1, OUT),
                               conv_bias[l].reshape(1, OUT))
        parts.append(_segment_scatter(s, mol_ids))
        x = act
        ns = st
    s = _fp_tc(x, ns, W_out[NL], b_out[NL].reshape(1, OUT))
    parts.append(_segment_scatter(s, mol_ids))
    return _fp_sum(parts)


# speedup vs baseline: 5.5479x; 1.1308x over previous
import jax
import jax.numpy as jnp
from jax import lax
from jax.experimental import pallas as pl
from jax.experimental.pallas import tpu as pltpu
from jax.experimental.pallas import tpu_sc as plsc

N_ATOM = 100000
ND = 25000
NB = 250000
M_MOL = 4096
NODE = 128
EDGE = 16
OUT = 128
NL = 2
DEGS = (1, 2, 3, 4)

NTILE = 32
T_MAIN = 776
C_MAIN = 96
N_CHUNK = 8
T_TAIL = 8
REM_TILES = 21
GOFF = (0, ND, 3 * ND, 6 * ND)


def _make_conv_body(apply_norm):
    def body(*args):
        if apply_norm:
            (x_hbm, aidx_hbm, ns_hbm, a_out,
             idxa0, idxa1, idxa2, idxa3, stga, tstg, nsv,
             gsem0, gsem1, osem0, osem1, tsem, rsem) = args
        else:
            (x_hbm, aidx_hbm, a_out,
             idxa0, idxa1, idxa2, idxa3, stga, tstg,
             gsem0, gsem1, osem0, osem1, tsem, rsem) = args
            nsv = None
        idxa = (idxa0, idxa1, idxa2, idxa3)
        gsem = (gsem0, gsem1)
        osem = (osem0, osem1)
        cid = lax.axis_index("c")
        sid = lax.axis_index("s")
        w = sid * 2 + cid

        if apply_norm:
            pltpu.sync_copy(ns_hbm, nsv)
            scs = [nsv[2, pl.ds(c * 16, 16)] for c in range(NODE // 16)]
            shs = [nsv[3, pl.ds(c * 16, 16)] for c in range(NODE // 16)]
        else:
            scs = shs = None

        for g, d in enumerate(DEGS):
            goff = GOFF[g]
            orow0 = g * ND
            row_base = orow0 + w * T_MAIN
            rbase = NTILE * T_MAIN

            for j in range(d):
                pltpu.sync_copy(
                    aidx_hbm.at[pl.ds(goff + j * ND + w * T_MAIN, T_MAIN)],
                    idxa[j].at[pl.ds(0, T_MAIN)])

            @pl.when(w < REM_TILES)
            def _():
                for j in range(d):
                    pltpu.sync_copy(
                        aidx_hbm.at[pl.ds(goff + j * ND + rbase + w * 8, 8)],
                        idxa[j].at[pl.ds(T_MAIN, 8)])

            def g_issue(b, col0, d=d):
                for j in range(d):
                    pltpu.async_copy(x_hbm.at[idxa[j].at[pl.ds(col0, C_MAIN)]],
                                     stga.at[b, j, :, :], gsem[b])

            def g_wait(b, d=d):
                for j in range(d):
                    pltpu.make_async_copy(
                        x_hbm.at[idxa[j].at[pl.ds(0, C_MAIN)]],
                        stga.at[b, j, :, :], gsem[b]).wait()

            def sum_rows(stg, nrow, d=d):
                @plsc.parallel_loop(0, nrow, step=2)
                def srow(r):
                    for rr in (r, r + 1):
                        for col in range(NODE // 16):
                            sl = pl.ds(col * 16, 16)
                            if apply_norm:
                                v = jnp.maximum(
                                    stg[0, rr, sl] * scs[col] + shs[col], 0.0)
                                for j in range(1, d):
                                    v = v + jnp.maximum(
                                        stg[j, rr, sl] * scs[col] + shs[col],
                                        0.0)
                            else:
                                v = stg[0, rr, sl]
                                for j in range(1, d):
                                    v = v + stg[j, rr, sl]
                            stg[0, rr, sl] = v

            def w_wait(b):
                pltpu.make_async_copy(stga.at[b, 0, :, :],
                                      a_out.at[pl.ds(0, C_MAIN), :],
                                      osem[b]).wait()

            tail_cps = []
            for j in range(d):
                tail_cps.append(pltpu.async_copy(
                    x_hbm.at[idxa[j].at[pl.ds(N_CHUNK * C_MAIN, 8)]],
                    tstg.at[j, pl.ds(0, 8), :], tsem))

            @pl.when(w < REM_TILES)
            def _():
                for j in range(d):
                    pltpu.async_copy(
                        x_hbm.at[idxa[j].at[pl.ds(T_MAIN, 8)]],
                        tstg.at[j, pl.ds(8, 8), :], rsem)

            g_issue(0, 0)

            def lbody(k, _):
                for b in (0, 1):
                    @pl.when(lax.rem(k, 2) == b)
                    def _(b=b):
                        bo = 1 - b
                        @pl.when(k >= 1)
                        def _():
                            w_wait(bo)
                        @pl.when(k + 1 < N_CHUNK)
                        def _():
                            g_issue(bo, (k + 1) * C_MAIN)
                        g_wait(b)
                        sum_rows(stga.at[b], C_MAIN)
                        pltpu.async_copy(
                            stga.at[b, 0, :, :],
                            a_out.at[pl.ds(row_base + k * C_MAIN, C_MAIN), :],
                            osem[b])
                return 0
            lax.fori_loop(0, N_CHUNK, lbody, 0, unroll=False)
            w_wait((N_CHUNK - 1) % 2)

            for cp in tail_cps:
                cp.wait()
            sum_rows(tstg, 8)
            pltpu.async_copy(tstg.at[0, pl.ds(0, 8), :],
                             a_out.at[pl.ds(row_base + N_CHUNK * C_MAIN, 8), :],
                             tsem).wait()

            @pl.when(w < REM_TILES)
            def _():
                for j in range(d):
                    pltpu.make_async_copy(
                        x_hbm.at[idxa[j].at[pl.ds(T_MAIN, 8)]],
                        tstg.at[j, pl.ds(8, 8), :], rsem).wait()
                @plsc.parallel_loop(8, 16)
                def srow(r):
                    for col in range(NODE // 16):
                        sl = pl.ds(col * 16, 16)
                        if apply_norm:
                            v = jnp.maximum(
                                tstg[0, r, sl] * scs[col] + shs[col], 0.0)
                            for j in range(1, d):
                                v = v + jnp.maximum(
                                    tstg[j, r, sl] * scs[col] + shs[col], 0.0)
                        else:
                            v = tstg[0, r, sl]
                            for j in range(1, d):
                                v = v + tstg[j, r, sl]
                        tstg[0, r, sl] = v
                pltpu.async_copy(tstg.at[0, pl.ds(8, 8), :],
                                 a_out.at[pl.ds(orow0 + rbase + w * 8, 8), :],
                                 tsem).wait()
    return body


def _conv_gather(x, aidx, ns=None):
    mesh = plsc.VectorSubcoreMesh(core_axis_name="c", subcore_axis_name="s",
                                  num_cores=2, num_subcores=16)
    apply_norm = ns is not None
    scratch = (
        [pltpu.VMEM((T_MAIN + 8,), jnp.int32) for _ in range(4)]
        + [pltpu.VMEM((2, 4, C_MAIN, NODE), jnp.float32),
           pltpu.VMEM((4, 16, NODE), jnp.float32)]
        + ([pltpu.VMEM((8, NODE), jnp.float32)] if apply_norm else [])
        + [pltpu.SemaphoreType.DMA for _ in range(6)])
    f = pl.kernel(
        _make_conv_body(apply_norm),
        out_type=jax.ShapeDtypeStruct((N_ATOM, NODE), jnp.float32),
        mesh=mesh,
        scratch_types=scratch,
    )
    return f(x, aidx, ns) if apply_norm else f(x, aidx)


def _bond_body(bond_hbm, bidx_hbm, bf_out,
               idxb0, idxb1, idxb2, idxb3, stgb, tstgb, bacc,
               gsem0, gsem1, osem0, osem1, tsem, rsem):
    idxb = (idxb0, idxb1, idxb2, idxb3)
    gsem = (gsem0, gsem1)
    osem = (osem0, osem1)
    cid = lax.axis_index("c")
    sid = lax.axis_index("s")
    w = sid * 2 + cid

    for g, d in enumerate(DEGS):
        goff = GOFF[g]
        orow0 = g * ND
        row_base = orow0 + w * T_MAIN
        rbase = NTILE * T_MAIN

        for j in range(d):
            pltpu.sync_copy(bidx_hbm.at[pl.ds(goff + j * ND + w * T_MAIN, T_MAIN)],
                            idxb[j].at[pl.ds(0, T_MAIN)])

        @pl.when(w < REM_TILES)
        def _():
            for j in range(d):
                pltpu.sync_copy(
                    bidx_hbm.at[pl.ds(goff + j * ND + rbase + w * 8, 8)],
                    idxb[j].at[pl.ds(T_MAIN, 8)])

        def g_issue(b, col0, d=d):
            for j in range(d):
                pltpu.async_copy(bond_hbm.at[idxb[j].at[pl.ds(col0, C_MAIN)]],
                                 stgb.at[b, j, :, :], gsem[b])

        def g_wait(b, d=d):
            for j in range(d):
                pltpu.make_async_copy(
                    bond_hbm.at[idxb[j].at[pl.ds(0, C_MAIN)]],
                    stgb.at[b, j, :, :], gsem[b]).wait()

        def sum_rows(stg, bac, nrow, d=d):
            @plsc.parallel_loop(0, nrow, step=2)
            def srow(r):
                for rr in (r, r + 1):
                    v = stg[0, rr, :]
                    for j in range(1, d):
                        v = v + stg[j, rr, :]
                    bac[pl.ds(rr * EDGE, EDGE)] = v

        def w_wait(b):
            pltpu.make_async_copy(bacc.at[b, pl.ds(0, C_MAIN * EDGE)],
                                  bf_out.at[pl.ds(0, C_MAIN * EDGE)],
                                  osem[b]).wait()

        tail_cps = []
        for j in range(d):
            tail_cps.append(pltpu.async_copy(
                bond_hbm.at[idxb[j].at[pl.ds(N_CHUNK * C_MAIN, 8)]],
                tstgb.at[j, pl.ds(0, 8), :], tsem))

        @pl.when(w < REM_TILES)
        def _():
            for j in range(d):
                pltpu.async_copy(
                    bond_hbm.at[idxb[j].at[pl.ds(T_MAIN, 8)]],
                    tstgb.at[j, pl.ds(8, 8), :], rsem)

        g_issue(0, 0)

        def lbody(k, _):
            for b in (0, 1):
                @pl.when(lax.rem(k, 2) == b)
                def _(b=b):
                    bo = 1 - b
                    @pl.when(k >= 1)
                    def _():
                        w_wait(bo)
                    @pl.when(k + 1 < N_CHUNK)
                    def _():
                        g_issue(bo, (k + 1) * C_MAIN)
                    g_wait(b)
                    sum_rows(stgb.at[b], bacc.at[b], C_MAIN)
                    pltpu.async_copy(
                        bacc.at[b, pl.ds(0, C_MAIN * EDGE)],
                        bf_out.at[pl.ds((row_base + k * C_MAIN) * EDGE,
                                        C_MAIN * EDGE)], osem[b])
            return 0
        lax.fori_loop(0, N_CHUNK, lbody, 0, unroll=False)
        w_wait((N_CHUNK - 1) % 2)

        for cp in tail_cps:
            cp.wait()
        sum_rows(tstgb, bacc.at[0], 8)
        pltpu.async_copy(bacc.at[0, pl.ds(0, 8 * EDGE)],
                         bf_out.at[pl.ds((row_base + N_CHUNK * C_MAIN) * EDGE,
                                         8 * EDGE)], tsem).wait()

        @pl.when(w < REM_TILES)
        def _():
            for j in range(d):
                pltpu.make_async_copy(
                    bond_hbm.at[idxb[j].at[pl.ds(T_MAIN, 8)]],
                    tstgb.at[j, pl.ds(8, 8), :], rsem).wait()
            @plsc.parallel_loop(8, 16, step=2)
            def srow(r):
                for rr in (r, r + 1):
                    v = tstgb[0, rr, :]
                    for j in range(1, d):
                        v = v + tstgb[j, rr, :]
                    bacc[0, pl.ds(rr * EDGE, EDGE)] = v
            pltpu.async_copy(bacc.at[0, pl.ds(8 * EDGE, 8 * EDGE)],
                             bf_out.at[pl.ds((orow0 + rbase + w * 8) * EDGE,
                                             8 * EDGE)], tsem).wait()


def _bond_gather(bond, bidx):
    mesh = plsc.VectorSubcoreMesh(core_axis_name="c", subcore_axis_name="s",
                                  num_cores=2, num_subcores=16)
    f = pl.kernel(
        _bond_body,
        out_type=jax.ShapeDtypeStruct((N_ATOM * EDGE,), jnp.float32),
        mesh=mesh,
        scratch_types=(
            [pltpu.VMEM((T_MAIN + 8,), jnp.int32) for _ in range(4)]
            + [
                pltpu.VMEM((2, 4, C_MAIN, EDGE), jnp.float32),
                pltpu.VMEM((4, 16, EDGE), jnp.float32),
                pltpu.VMEM((2, C_MAIN * EDGE), jnp.float32),
            ]
            + [pltpu.SemaphoreType.DMA for _ in range(6)]),
        compiler_params=pltpu.CompilerParams(use_tc_tiling_on_sc=False),
    )
    return f(bond, bidx)


SEG_T = 3128
SEG_C = 128


def _seg_body(s_hbm, ids_hbm, out_hbm, stg, idx0, idx1, idx2, idx56, idx8,
              zbuf, acc, lsem0, lsem1, lsem2):
    idxb = (idx0, idx1, idx2)
    lsem = (lsem0, lsem1, lsem2)
    cid = lax.axis_index("c")
    sid = lax.axis_index("s")
    base = cid * (N_ATOM // 2) + sid * SEG_T

    def ld_issue(b, k):
        r0 = base + k * SEG_C
        pltpu.async_copy(ids_hbm.at[pl.ds(r0, SEG_C)], idxb[b], lsem[b])
        pltpu.async_copy(s_hbm.at[pl.ds(r0, SEG_C), :], stg.at[b], lsem[b])

    def ld_wait(b):
        pltpu.make_async_copy(ids_hbm.at[pl.ds(0, SEG_C)], idxb[b],
                              lsem[b]).wait()
        pltpu.make_async_copy(s_hbm.at[pl.ds(0, SEG_C), :], stg.at[b],
                              lsem[b]).wait()

    ld_issue(0, 0)
    ld_issue(1, 1)

    @plsc.parallel_loop(0, M_MOL // 16)
    def zrow(r):
        for col in range(NODE // 16):
            zbuf[r, pl.ds(col * 16, 16)] = jnp.zeros((16,), jnp.float32)
    pltpu.sync_copy(zbuf, acc.at[pl.ds(sid * (M_MOL // 16), M_MOL // 16), :])
    plsc.subcore_barrier()

    def lbody(k, _):
        for b in (0, 1, 2):
            @pl.when(lax.rem(k, 3) == b)
            def _(b=b):
                @pl.when(k + 2 < 24)
                def _():
                    ld_issue((b + 2) % 3, k + 2)
                ld_wait(b)
                pltpu.sync_copy(stg.at[b], acc.at[idxb[b]], add=True)
        return 0
    lax.fori_loop(0, 24, lbody, 0, unroll=False)

    r0 = base + 24 * SEG_C

    @pl.when(sid < 15)
    def _():
        pltpu.sync_copy(ids_hbm.at[pl.ds(r0, 56)], idx56)
        pltpu.sync_copy(s_hbm.at[pl.ds(r0, 56), :], stg.at[0, pl.ds(0, 56), :])
        pltpu.sync_copy(stg.at[0, pl.ds(0, 56), :], acc.at[idx56], add=True)

    @pl.when(sid == 15)
    def _():
        pltpu.sync_copy(ids_hbm.at[pl.ds(r0, 8)], idx8)
        pltpu.sync_copy(s_hbm.at[pl.ds(r0, 8), :], stg.at[0, pl.ds(0, 8), :])
        pltpu.sync_copy(stg.at[0, pl.ds(0, 8), :], acc.at[idx8], add=True)

    plsc.subcore_barrier()
    pltpu.sync_copy(acc.at[pl.ds(sid * 256, 256), :],
                    out_hbm.at[cid, pl.ds(sid * 256, 256), :])


def _segment_scatter(s, mol_ids):
    mesh = plsc.VectorSubcoreMesh(core_axis_name="c", subcore_axis_name="s",
                                  num_cores=2, num_subcores=16)
    f = pl.kernel(
        _seg_body,
        out_type=jax.ShapeDtypeStruct((2, M_MOL, OUT), jnp.float32),
        mesh=mesh,
        scratch_types=[
            pltpu.VMEM((3, SEG_C, OUT), jnp.float32),
            pltpu.VMEM((SEG_C,), jnp.int32),
            pltpu.VMEM((SEG_C,), jnp.int32),
            pltpu.VMEM((SEG_C,), jnp.int32),
            pltpu.VMEM((56,), jnp.int32),
            pltpu.VMEM((8,), jnp.int32),
            pltpu.VMEM((M_MOL // 16, OUT), jnp.float32),
            pltpu.VMEM_SHARED((M_MOL, OUT), jnp.float32),
            pltpu.SemaphoreType.DMA,
            pltpu.SemaphoreType.DMA,
            pltpu.SemaphoreType.DMA,
        ],
    )
    return f(s, mol_ids)


BLK = 5000
NBLK = N_ATOM // BLK
BPG = ND // BLK


def _make_layer_body(apply_norm):
    def body(*args):
        if apply_norm:
            (x_ref, ns_ref, a_ref, bf_ref, wda_ref, kb_ref, ws_ref, wo_ref,
             bo_ref, cb_ref, s_ref, act_ref, st_ref) = args
        else:
            (x_ref, a_ref, bf_ref, wda_ref, kb_ref, ws_ref, wo_ref,
             bo_ref, cb_ref, s_ref, act_ref, st_ref) = args
        i = pl.program_id(0)
        x = x_ref[...]
        if apply_norm:
            x = jnp.maximum(x * ns_ref[2][None, :] + ns_ref[3][None, :], 0.0)
        y = (jnp.dot(x, wo_ref[...], preferred_element_type=jnp.float32)
             + bo_ref[0][None, :])
        y = y - jnp.max(y, axis=1, keepdims=True)
        e = jnp.exp(y)
        s_ref[...] = e / jnp.sum(e, axis=1, keepdims=True)

        bf = bf_ref[...].reshape(BLK // 8, NODE)
        bp = jnp.dot(bf, kb_ref[0], preferred_element_type=jnp.float32)
        act = (jnp.dot(x, ws_ref[...], preferred_element_type=jnp.float32)
               + jnp.dot(a_ref[...], wda_ref[0],
                         preferred_element_type=jnp.float32)
               + bp.reshape(BLK, OUT)
               + cb_ref[0][None, :])
        act_ref[...] = act

        @pl.when(i == 0)
        def _():
            st_ref[...] = jnp.zeros_like(st_ref)
        st_ref[0, :] = st_ref[0, :] + jnp.sum(act, axis=0)
        st_ref[1, :] = st_ref[1, :] + jnp.sum(act * act, axis=0)

        @pl.when(i == NBLK - 1)
        def _():
            mean = st_ref[0, :] * (1.0 / N_ATOM)
            var = st_ref[1, :] * (1.0 / N_ATOM) - mean * mean
            inv = lax.rsqrt(var + 1e-5)
            st_ref[2, :] = inv
            st_ref[3, :] = -mean * inv
    return body


def _layer_tc(x, ns, a_sum, bfold, wda, kb, wself, wout, bout, cbias):
    apply_norm = ns is not None
    blockspec_x = pl.BlockSpec((BLK, NODE), lambda i: (i, 0))
    in_specs = [blockspec_x]
    args = [x]
    if apply_norm:
        in_specs.append(pl.BlockSpec((8, NODE), lambda i: (0, 0)))
        args.append(ns)
    in_specs += [
        pl.BlockSpec((BLK, NODE), lambda i: (i, 0)),
        pl.BlockSpec((BLK // 8, 1, NODE), lambda i: (i, 0, 0)),
        pl.BlockSpec((1, NODE, OUT), lambda i: (i // BPG, 0, 0)),
        pl.BlockSpec((1, NODE, 8 * OUT), lambda i: (i // BPG, 0, 0)),
        pl.BlockSpec((NODE, OUT), lambda i: (0, 0)),
        pl.BlockSpec((NODE, OUT), lambda i: (0, 0)),
        pl.BlockSpec((1, OUT), lambda i: (0, 0)),
        pl.BlockSpec((1, OUT), lambda i: (0, 0)),
    ]
    args += [a_sum, bfold, wda, kb, wself, wout, bout, cbias]
    return pl.pallas_call(
        _make_layer_body(apply_norm),
        grid=(NBLK,),
        in_specs=in_specs,
        out_specs=[
            pl.BlockSpec((BLK, OUT), lambda i: (i, 0)),
            pl.BlockSpec((BLK, OUT), lambda i: (i, 0)),
            pl.BlockSpec((8, OUT), lambda i: (0, 0)),
        ],
        out_shape=[
            jax.ShapeDtypeStruct((N_ATOM, OUT), jnp.float32),
            jax.ShapeDtypeStruct((N_ATOM, OUT), jnp.float32),
            jax.ShapeDtypeStruct((8, OUT), jnp.float32),
        ],
        compiler_params=pltpu.CompilerParams(dimension_semantics=("arbitrary",)),
    )(*args)


def _fp_body(x_ref, ns_ref, wo_ref, bo_ref, s_ref):
    x = x_ref[...]
    x = jnp.maximum(x * ns_ref[2][None, :] + ns_ref[3][None, :], 0.0)
    y = (jnp.dot(x, wo_ref[...], preferred_element_type=jnp.float32)
         + bo_ref[0][None, :])
    y = y - jnp.max(y, axis=1, keepdims=True)
    e = jnp.exp(y)
    s_ref[...] = e / jnp.sum(e, axis=1, keepdims=True)


def _fp_tc(x, ns, wout, bout):
    return pl.pallas_call(
        _fp_body,
        grid=(NBLK,),
        in_specs=[
            pl.BlockSpec((BLK, NODE), lambda i: (i, 0)),
            pl.BlockSpec((8, NODE), lambda i: (0, 0)),
            pl.BlockSpec((NODE, OUT), lambda i: (0, 0)),
            pl.BlockSpec((1, OUT), lambda i: (0, 0)),
        ],
        out_specs=pl.BlockSpec((BLK, OUT), lambda i: (i, 0)),
        out_shape=jax.ShapeDtypeStruct((N_ATOM, OUT), jnp.float32),
    )(x, ns, wout, bout)


def _fpsum_body(p0_ref, p1_ref, p2_ref, o_ref):
    o_ref[...] = ((p0_ref[0] + p0_ref[1]) + (p1_ref[0] + p1_ref[1])
                  + (p2_ref[0] + p2_ref[1]))


def _fp_sum(parts):
    return pl.pallas_call(
        _fpsum_body,
        out_shape=jax.ShapeDtypeStruct((M_MOL, OUT), jnp.float32),
    )(*parts)


def kernel(atom_repr, bond_repr, atom_nbr_d1, atom_nbr_d2, atom_nbr_d3,
           atom_nbr_d4, bond_nbr_d1, bond_nbr_d2, bond_nbr_d3, bond_nbr_d4,
           mol_ids, W_deg, W_self, conv_bias, W_out, b_out):
    atom_nbrs = (atom_nbr_d1, atom_nbr_d2, atom_nbr_d3, atom_nbr_d4)
    bond_nbrs = (bond_nbr_d1, bond_nbr_d2, bond_nbr_d3, bond_nbr_d4)
    aidx = jnp.concatenate([a.T.reshape(-1) for a in atom_nbrs])
    bidx = jnp.concatenate([b.T.reshape(-1) for b in bond_nbrs])

    b_flat = _bond_gather(bond_repr, bidx)
    bfold = b_flat.reshape(N_ATOM // 8, 1, 8 * EDGE)

    kbs = []
    for l in range(NL):
        wdb = W_deg[l][:, NODE:, :]
        kb = jnp.zeros((4, NODE, 8 * OUT), dtype=jnp.float32)
        for i in range(8):
            kb = kb.at[:, i * EDGE:(i + 1) * EDGE, i * OUT:(i + 1) * OUT].set(wdb)
        kbs.append(kb)

    x = atom_repr
    ns = None
    parts = []
    for l in range(NL):
        a_sum = _conv_gather(x, aidx, ns)
        s, act, st = _layer_tc(x, ns, a_sum, bfold, W_deg[l][:, :NODE, :],
                               kbs[l], W_self[l], W_out[l],
                               b_out[l].reshape(1, OUT),
                               conv_bias[l].reshape(1, OUT))
        parts.append(_segment_scatter(s, mol_ids))
        x = act
        ns = st
    s = _fp_tc(x, ns, W_out[NL], b_out[NL].reshape(1, OUT))
    parts.append(_segment_scatter(s, mol_ids))
    return _fp_sum(parts)
